# baseline (speedup 1.0000x reference)
import functools
import jax
import jax.numpy as jnp
from jax import lax
from jax.experimental import pallas as pl
from jax.experimental.pallas import tpu as pltpu
from jax.experimental.pallas import tpu_sc as plsc

N = 10000
E = 320000
D = 128
NB = 16
NC = 2
NS = 16
NW = NC * NS
L = 16

BE = 2560
C2 = 400
C4 = 512
FPT = 4

_NEG_INF = float("-inf")


def _proj_body(x_ref, wex_ref, wnx_ref, q_ref, pT_ref):
    xb = x_ref[...]
    q_ref[...] = lax.dot_general(xb, wex_ref[...], (((1,), (1,)), ((), ())),
                                 preferred_element_type=jnp.float32)
    pT_ref[...] = lax.dot_general(wnx_ref[...], xb, (((1,), (1,)), ((), ())),
                                  preferred_element_type=jnp.float32)


def _k1a(x, Wex, Wnx):
    return pl.pallas_call(
        _proj_body,
        out_shape=[
            jax.ShapeDtypeStruct((N, D), jnp.float32),
            jax.ShapeDtypeStruct((D, N), jnp.float32),
        ],
    )(x, Wex, Wnx)


def _r_body(ea_ref, wea_ref, be_ref, r_ref):
    r_ref[...] = lax.dot_general(ea_ref[...], wea_ref[...],
                                 (((1,), (1,)), ((), ())),
                                 preferred_element_type=jnp.float32) + be_ref[...]


def _k1b(edge_attr, Wea, be2d):
    return pl.pallas_call(
        _r_body,
        grid=(E // BE,),
        in_specs=[
            pl.BlockSpec((BE, 16), lambda i: (i, 0)),
            pl.BlockSpec((D, 16), lambda i: (0, 0)),
            pl.BlockSpec((1, D), lambda i: (0, 0)),
        ],
        out_specs=pl.BlockSpec((BE, D), lambda i: (i, 0)),
        out_shape=jax.ShapeDtypeStruct((E, D), jnp.float32),
    )(edge_attr, Wea, be2d)


def _hT_body(e_ref, wne_ref, bn_ref, hT_ref):
    hT_ref[...] = lax.dot_general(wne_ref[...], e_ref[...],
                                  (((1,), (1,)), ((), ())),
                                  preferred_element_type=jnp.float32) + bn_ref[...]


def _k3(e, Wne, bn2d):
    return pl.pallas_call(
        _hT_body,
        grid=(E // BE,),
        in_specs=[
            pl.BlockSpec((BE, D), lambda i: (i, 0)),
            pl.BlockSpec((D, D), lambda i: (0, 0)),
            pl.BlockSpec((D, 1), lambda i: (0, 0)),
        ],
        out_specs=pl.BlockSpec((D, BE), lambda i: (0, i)),
        out_shape=jax.ShapeDtypeStruct((D, E), jnp.float32),
    )(e, Wne, bn2d)


def _final_body(aggT_ref, b2d_ref, glob_ref, w2a_ref, b2_ref, w2g_ref,
                wgg_ref, wgm_ref, bg_ref, xn_ref, u_ref):
    aggT = aggT_ref[...]
    xb = lax.dot_general(aggT, w2a_ref[...], (((0,), (1,)), ((), ())),
                         preferred_element_type=jnp.float32)
    oh = (b2d_ref[...] == lax.broadcasted_iota(jnp.int32, (N, NB), 1))
    oh = oh.astype(jnp.float32)
    G2 = lax.dot_general(glob_ref[...], w2g_ref[...], (((1,), (1,)), ((), ())),
                         preferred_element_type=jnp.float32)
    xn = xb + lax.dot_general(oh, G2, (((1,), (0,)), ((), ())),
                              preferred_element_type=jnp.float32) + b2_ref[...]
    xn = jnp.maximum(xn, 0.0)
    xn_ref[...] = xn

    S = lax.dot_general(oh, xn, (((0,), (0,)), ((), ())),
                        preferred_element_type=jnp.float32)
    cnt = lax.dot_general(oh, jnp.ones((N, D), jnp.float32),
                          (((0,), (0,)), ((), ())),
                          preferred_element_type=jnp.float32)
    mean = S / jnp.maximum(cnt, 1.0)
    u = (lax.dot_general(glob_ref[...], wgg_ref[...],
                         (((1,), (1,)), ((), ())),
                         preferred_element_type=jnp.float32)
         + lax.dot_general(mean, wgm_ref[...], (((1,), (1,)), ((), ())),
                           preferred_element_type=jnp.float32)
         + bg_ref[...])
    u_ref[...] = jnp.maximum(u, 0.0)


def _k5(aggT, batch2d, glob, W2a, b22d, W2g, Wgg, Wgm, bg2d):
    return pl.pallas_call(
        _final_body,
        out_shape=[
            jax.ShapeDtypeStruct((N, D), jnp.float32),
            jax.ShapeDtypeStruct((NB, 32), jnp.float32),
        ],
    )(aggT, batch2d, glob, W2a, b22d, W2g, Wgg, Wgm, bg2d)


def _sc_mesh():
    return plsc.VectorSubcoreMesh(core_axis_name="c", subcore_axis_name="s")


_SC_PARAMS = pltpu.CompilerParams(needs_layout_passes=False,
                                  use_tc_tiling_on_sc=False)


def _edge_sc_body(q_hbm, row_hbm, r_hbm, e_hbm, idx_v, qrows_v, rbuf_v, sem):
    wid = lax.axis_index("s") * NC + lax.axis_index("c")
    base0 = wid * (E // NW)
    nchunks = (E // NW) // C2

    def chunk(ci, carry):
        base = base0 + ci * C2
        pltpu.sync_copy(row_hbm.at[pl.ds(base, C2)], idx_v)
        cp = pltpu.async_copy(q_hbm.at[idx_v], qrows_v, sem)
        pltpu.sync_copy(r_hbm.at[pl.ds(base, C2)], rbuf_v)
        cp.wait()

        def rowloop(i, c2):
            for j in range(D // L):
                sl = pl.ds(j * L, L)
                v = qrows_v[i, sl] + rbuf_v[i, sl]
                rbuf_v[i, sl] = jnp.maximum(v, 0.0)
            return c2

        lax.fori_loop(0, C2, rowloop, 0)
        pltpu.sync_copy(rbuf_v, e_hbm.at[pl.ds(base, C2)])
        return carry

    lax.fori_loop(0, nchunks, chunk, 0)


def _k2(q, row, r):
    fn = pl.kernel(
        _edge_sc_body,
        out_type=jax.ShapeDtypeStruct((E, D), jnp.float32),
        mesh=_sc_mesh(),
        compiler_params=_SC_PARAMS,
        scratch_types=[
            pltpu.VMEM((C2,), jnp.int32),
            pltpu.VMEM((C2, D), jnp.float32),
            pltpu.VMEM((C2, D), jnp.float32),
            pltpu.SemaphoreType.DMA,
        ],
    )
    return fn(q, row, r)


def _segmax_body(hT_hbm, row_hbm, col_hbm, pT_hbm, aggT_hbm,
                 p_v, agg_v, hbuf_v, rowb_v, colb_v):
    wid = lax.axis_index("s") * NC + lax.axis_index("c")
    f0 = wid * FPT

    for f in range(FPT):
        pltpu.sync_copy(pT_hbm.at[f0 + f], p_v.at[pl.ds(f * N, N)])

    neg = jnp.full((L,), _NEG_INF, jnp.float32)

    def initloop(i, carry):
        agg_v[pl.ds(i * L, L)] = neg
        return carry

    lax.fori_loop(0, (FPT * N) // L, initloop, 0)

    nchunks = E // C4

    def chunk(ci, carry):
        base = ci * C4
        pltpu.sync_copy(row_hbm.at[pl.ds(base, C4)], rowb_v)
        pltpu.sync_copy(col_hbm.at[pl.ds(base, C4)], colb_v)
        for f in range(FPT):
            pltpu.sync_copy(hT_hbm.at[f0 + f, pl.ds(base, C4)],
                            hbuf_v.at[pl.ds(f * C4, C4)])

        def group(k, c2):
            sl = pl.ds(k * L, L)
            rv = rowb_v[sl]
            cv = colb_v[sl]
            for f in range(FPT):
                hv = hbuf_v[pl.ds(f * C4 + k * L, L)] \
                    + plsc.load_gather(p_v, [cv + (f * N)])
                ri = rv + (f * N)

                def cond(m):
                    return jnp.any(m)

                def body(m):
                    cur = plsc.load_gather(agg_v, [ri])
                    val = jnp.maximum(cur, hv)
                    plsc.store_scatter(agg_v, [ri], val, mask=m)
                    chk = plsc.load_gather(agg_v, [ri])
                    return m & (chk < hv)

                lax.while_loop(cond, body, jnp.ones((L,), jnp.bool_))
            return c2

        lax.fori_loop(0, C4 // L, group, 0)
        return carry

    lax.fori_loop(0, nchunks, chunk, 0)

    def fixloop(i, carry):
        sl = pl.ds(i * L, L)
        v = agg_v[sl]
        ok = (v - v) == 0.0
        agg_v[sl] = jnp.where(ok, v, 0.0)
        return carry

    lax.fori_loop(0, (FPT * N) // L, fixloop, 0)
    for f in range(FPT):
        pltpu.sync_copy(agg_v.at[pl.ds(f * N, N)], aggT_hbm.at[f0 + f])


def _k4(hT, row, col, pT):
    fn = pl.kernel(
        _segmax_body,
        out_type=jax.ShapeDtypeStruct((D, N), jnp.float32),
        mesh=_sc_mesh(),
        compiler_params=_SC_PARAMS,
        scratch_types=[
            pltpu.VMEM((FPT * N,), jnp.float32),
            pltpu.VMEM((FPT * N,), jnp.float32),
            pltpu.VMEM((FPT * C4,), jnp.float32),
            pltpu.VMEM((C4,), jnp.int32),
            pltpu.VMEM((C4,), jnp.int32),
        ],
    )
    return fn(hT, row, col, pT)


def kernel(x, edge_index, edge_attr, glob, batch,
           W_edge, b_edge, W_node, b_node, W_node2, b_node2, W_glob, b_glob):
    row = edge_index[0]
    col = edge_index[1]
    Wex, Wea = W_edge[:, :D], W_edge[:, D:]
    Wnx, Wne = W_node[:, :D], W_node[:, D:]
    W2a, W2g = W_node2[:, :D], W_node2[:, D:]
    Wgg, Wgm = W_glob[:, :32], W_glob[:, 32:]
    be2d = b_edge.reshape(1, D)
    bn2d = b_node.reshape(D, 1)
    b22d = b_node2.reshape(1, D)
    bg2d = b_glob.reshape(1, 32)
    batch2d = batch.reshape(N, 1)

    q, pT = _k1a(x, Wex, Wnx)
    r = _k1b(edge_attr, Wea, be2d)
    e = _k2(q, row, r)
    hT = _k3(e, Wne, bn2d)
    aggT = _k4(hT, row, col, pT)
    xn, u = _k5(aggT, batch2d, glob, W2a, b22d, W2g, Wgg, Wgm, bg2d)
    return (xn, e, u)


# speedup vs baseline: 3.0072x; 3.0072x over previous
import functools
import jax
import jax.numpy as jnp
from jax import lax
from jax.experimental import pallas as pl
from jax.experimental.pallas import tpu as pltpu
from jax.experimental.pallas import tpu_sc as plsc

N = 10000
E = 320000
D = 128
NB = 16
NC = 2
NS = 16
NW = NC * NS
L = 16

BE = 2560
C2 = 400
C4 = 640
FPT = 4

_NEG_INF = float("-inf")


def _proj_body(x_ref, wex_ref, wnx_ref, q_ref, pT_ref):
    xb = x_ref[...]
    q_ref[...] = lax.dot_general(xb, wex_ref[...], (((1,), (1,)), ((), ())),
                                 preferred_element_type=jnp.float32)
    pT_ref[...] = lax.dot_general(wnx_ref[...], xb, (((1,), (1,)), ((), ())),
                                  preferred_element_type=jnp.float32)


def _k1a(x, Wex, Wnx):
    return pl.pallas_call(
        _proj_body,
        out_shape=[
            jax.ShapeDtypeStruct((N, D), jnp.float32),
            jax.ShapeDtypeStruct((D, N), jnp.float32),
        ],
    )(x, Wex, Wnx)


def _r_body(ea_ref, wea_ref, be_ref, r_ref):
    r_ref[...] = lax.dot_general(ea_ref[...], wea_ref[...],
                                 (((1,), (1,)), ((), ())),
                                 preferred_element_type=jnp.float32) + be_ref[...]


def _k1b(edge_attr, Wea, be2d):
    return pl.pallas_call(
        _r_body,
        grid=(E // BE,),
        in_specs=[
            pl.BlockSpec((BE, 16), lambda i: (i, 0)),
            pl.BlockSpec((D, 16), lambda i: (0, 0)),
            pl.BlockSpec((1, D), lambda i: (0, 0)),
        ],
        out_specs=pl.BlockSpec((BE, D), lambda i: (i, 0)),
        out_shape=jax.ShapeDtypeStruct((E, D), jnp.float32),
    )(edge_attr, Wea, be2d)


def _hT_body(e_ref, wne_ref, bn_ref, hT_ref):
    hT_ref[...] = lax.dot_general(wne_ref[...], e_ref[...],
                                  (((1,), (1,)), ((), ())),
                                  preferred_element_type=jnp.float32) + bn_ref[...]


def _k3(e, Wne, bn2d):
    return pl.pallas_call(
        _hT_body,
        grid=(E // BE,),
        in_specs=[
            pl.BlockSpec((BE, D), lambda i: (i, 0)),
            pl.BlockSpec((D, D), lambda i: (0, 0)),
            pl.BlockSpec((D, 1), lambda i: (0, 0)),
        ],
        out_specs=pl.BlockSpec((D, BE), lambda i: (0, i)),
        out_shape=jax.ShapeDtypeStruct((D, E), jnp.float32),
    )(e, Wne, bn2d)


def _final_body(aggT_ref, b2d_ref, glob_ref, w2a_ref, b2_ref, w2g_ref,
                wgg_ref, wgm_ref, bg_ref, xn_ref, u_ref):
    aggT = aggT_ref[...]
    xb = lax.dot_general(aggT, w2a_ref[...], (((0,), (1,)), ((), ())),
                         preferred_element_type=jnp.float32)
    oh = (b2d_ref[...] == lax.broadcasted_iota(jnp.int32, (N, NB), 1))
    oh = oh.astype(jnp.float32)
    G2 = lax.dot_general(glob_ref[...], w2g_ref[...], (((1,), (1,)), ((), ())),
                         preferred_element_type=jnp.float32)
    xn = xb + lax.dot_general(oh, G2, (((1,), (0,)), ((), ())),
                              preferred_element_type=jnp.float32) + b2_ref[...]
    xn = jnp.maximum(xn, 0.0)
    xn_ref[...] = xn

    S = lax.dot_general(oh, xn, (((0,), (0,)), ((), ())),
                        preferred_element_type=jnp.float32)
    cnt = lax.dot_general(oh, jnp.ones((N, D), jnp.float32),
                          (((0,), (0,)), ((), ())),
                          preferred_element_type=jnp.float32)
    mean = S / jnp.maximum(cnt, 1.0)
    u = (lax.dot_general(glob_ref[...], wgg_ref[...],
                         (((1,), (1,)), ((), ())),
                         preferred_element_type=jnp.float32)
         + lax.dot_general(mean, wgm_ref[...], (((1,), (1,)), ((), ())),
                           preferred_element_type=jnp.float32)
         + bg_ref[...])
    u_ref[...] = jnp.maximum(u, 0.0)


def _k5(aggT, batch2d, glob, W2a, b22d, W2g, Wgg, Wgm, bg2d):
    return pl.pallas_call(
        _final_body,
        out_shape=[
            jax.ShapeDtypeStruct((N, D), jnp.float32),
            jax.ShapeDtypeStruct((NB, 32), jnp.float32),
        ],
    )(aggT, batch2d, glob, W2a, b22d, W2g, Wgg, Wgm, bg2d)


def _sc_mesh():
    return plsc.VectorSubcoreMesh(core_axis_name="c", subcore_axis_name="s")


_SC_PARAMS = pltpu.CompilerParams(needs_layout_passes=False,
                                  use_tc_tiling_on_sc=False)


def _edge_sc_body(q_hbm, row_hbm, r_hbm, e_hbm, idx_v, qrows_v, rbuf_v, sem):
    wid = lax.axis_index("s") * NC + lax.axis_index("c")
    base0 = wid * (E // NW)
    nchunks = (E // NW) // C2

    def chunk(ci, carry):
        base = base0 + ci * C2
        pltpu.sync_copy(row_hbm.at[pl.ds(base, C2)], idx_v)
        cp = pltpu.async_copy(q_hbm.at[idx_v], qrows_v, sem)
        pltpu.sync_copy(r_hbm.at[pl.ds(base, C2)], rbuf_v)
        cp.wait()

        def rowloop(i, c2):
            for j in range(D // L):
                sl = pl.ds(j * L, L)
                v = qrows_v[i, sl] + rbuf_v[i, sl]
                rbuf_v[i, sl] = jnp.maximum(v, 0.0)
            return c2

        lax.fori_loop(0, C2, rowloop, 0)
        pltpu.sync_copy(rbuf_v, e_hbm.at[pl.ds(base, C2)])
        return carry

    lax.fori_loop(0, nchunks, chunk, 0)


def _k2(q, row, r):
    fn = pl.kernel(
        _edge_sc_body,
        out_type=jax.ShapeDtypeStruct((E, D), jnp.float32),
        mesh=_sc_mesh(),
        compiler_params=_SC_PARAMS,
        scratch_types=[
            pltpu.VMEM((C2,), jnp.int32),
            pltpu.VMEM((C2, D), jnp.float32),
            pltpu.VMEM((C2, D), jnp.float32),
            pltpu.SemaphoreType.DMA,
        ],
    )
    return fn(q, row, r)


def _segmax_body(hT_hbm, row_hbm, col_hbm, pT_hbm, aggT_hbm,
                 p_v, agg_v, tmp_v,
                 hbuf0, rowb0, colb0, hbuf1, rowb1, colb1, sem0, sem1):
    wid = lax.axis_index("s") * NC + lax.axis_index("c")
    f0 = wid * FPT
    slots = ((hbuf0, rowb0, colb0, sem0), (hbuf1, rowb1, colb1, sem1))
    nchunks = E // C4

    def issue(ci, slot):
        hbuf, rowb, colb, sem = slots[slot]
        base = ci * C4
        pltpu.async_copy(row_hbm.at[pl.ds(base, C4)], rowb, sem)
        pltpu.async_copy(col_hbm.at[pl.ds(base, C4)], colb, sem)
        for f in range(FPT):
            pltpu.async_copy(hT_hbm.at[f0 + f, pl.ds(base, C4)],
                             hbuf.at[pl.ds(f * C4, C4)], sem)

    def wait(ci, slot):
        hbuf, rowb, colb, sem = slots[slot]
        base = ci * C4
        pltpu.make_async_copy(row_hbm.at[pl.ds(base, C4)], rowb, sem).wait()
        pltpu.make_async_copy(col_hbm.at[pl.ds(base, C4)], colb, sem).wait()
        for f in range(FPT):
            pltpu.make_async_copy(hT_hbm.at[f0 + f, pl.ds(base, C4)],
                                  hbuf.at[pl.ds(f * C4, C4)], sem).wait()

    issue(0, 0)
    issue(1, 1)

    for f in range(FPT):
        pltpu.sync_copy(pT_hbm.at[f0 + f], p_v.at[pl.ds(f * N, N)])

    neg = jnp.full((L,), _NEG_INF, jnp.float32)

    def initloop(i, carry):
        agg_v[pl.ds(i * L, L)] = neg
        return carry

    lax.fori_loop(0, (FPT * N) // L, initloop, 0)

    ids = lax.broadcasted_iota(jnp.int32, (L,), 0)

    def process(hbuf, rowb, colb):
        def group(k, c2):
            sl = pl.ds(k * L, L)
            rv = rowb[sl]
            cv = colb[sl]
            plsc.store_scatter(tmp_v, [rv], ids)
            got = plsc.load_gather(tmp_v, [rv])
            has_dup = jnp.any(got != ids)
            hvs = []
            for f in range(FPT):
                hvs.append(hbuf[pl.ds(f * C4 + k * L, L)]
                           + plsc.load_gather(p_v, [cv + (f * N)]))

            @pl.when(jnp.logical_not(has_dup))
            def _():
                for f in range(FPT):
                    ri = rv + (f * N)
                    cur = plsc.load_gather(agg_v, [ri])
                    plsc.store_scatter(agg_v, [ri], jnp.maximum(cur, hvs[f]))

            @pl.when(has_dup)
            def _():
                for f in range(FPT):
                    ri = rv + (f * N)
                    hv = hvs[f]

                    def cond(m):
                        return jnp.any(m)

                    def body(m):
                        cur = plsc.load_gather(agg_v, [ri])
                        val = jnp.maximum(cur, hv)
                        plsc.store_scatter(agg_v, [ri], val, mask=m)
                        chk = plsc.load_gather(agg_v, [ri])
                        return m & (chk < hv)

                    lax.while_loop(cond, body, jnp.ones((L,), jnp.bool_))
            return c2

        lax.fori_loop(0, C4 // L, group, 0)

    def pair(i, carry):
        for slot in (0, 1):
            ci = 2 * i + slot
            wait(ci, slot)
            hbuf, rowb, colb, _ = slots[slot]
            process(hbuf, rowb, colb)

            @pl.when(ci + 2 < nchunks)
            def _():
                issue(ci + 2, slot)
        return carry

    lax.fori_loop(0, nchunks // 2, pair, 0)

    def fixloop(i, carry):
        sl = pl.ds(i * L, L)
        v = agg_v[sl]
        ok = (v - v) == 0.0
        agg_v[sl] = jnp.where(ok, v, 0.0)
        return carry

    lax.fori_loop(0, (FPT * N) // L, fixloop, 0)
    for f in range(FPT):
        pltpu.sync_copy(agg_v.at[pl.ds(f * N, N)], aggT_hbm.at[f0 + f])


def _k4(hT, row, col, pT):
    fn = pl.kernel(
        _segmax_body,
        out_type=jax.ShapeDtypeStruct((D, N), jnp.float32),
        mesh=_sc_mesh(),
        compiler_params=_SC_PARAMS,
        scratch_types=[
            pltpu.VMEM((FPT * N,), jnp.float32),
            pltpu.VMEM((FPT * N,), jnp.float32),
            pltpu.VMEM((N,), jnp.int32),
            pltpu.VMEM((FPT * C4,), jnp.float32),
            pltpu.VMEM((C4,), jnp.int32),
            pltpu.VMEM((C4,), jnp.int32),
            pltpu.VMEM((FPT * C4,), jnp.float32),
            pltpu.VMEM((C4,), jnp.int32),
            pltpu.VMEM((C4,), jnp.int32),
            pltpu.SemaphoreType.DMA,
            pltpu.SemaphoreType.DMA,
        ],
    )
    return fn(hT, row, col, pT)


def kernel(x, edge_index, edge_attr, glob, batch,
           W_edge, b_edge, W_node, b_node, W_node2, b_node2, W_glob, b_glob):
    row = edge_index[0]
    col = edge_index[1]
    Wex, Wea = W_edge[:, :D], W_edge[:, D:]
    Wnx, Wne = W_node[:, :D], W_node[:, D:]
    W2a, W2g = W_node2[:, :D], W_node2[:, D:]
    Wgg, Wgm = W_glob[:, :32], W_glob[:, 32:]
    be2d = b_edge.reshape(1, D)
    bn2d = b_node.reshape(D, 1)
    b22d = b_node2.reshape(1, D)
    bg2d = b_glob.reshape(1, 32)
    batch2d = batch.reshape(N, 1)

    q, pT = _k1a(x, Wex, Wnx)
    r = _k1b(edge_attr, Wea, be2d)
    e = _k2(q, row, r)
    hT = _k3(e, Wne, bn2d)
    aggT = _k4(hT, row, col, pT)
    xn, u = _k5(aggT, batch2d, glob, W2a, b22d, W2g, Wgg, Wgm, bg2d)
    return (xn, e, u)


# speedup vs baseline: 3.0774x; 1.0233x over previous
import functools
import jax
import jax.numpy as jnp
from jax import lax
from jax.experimental import pallas as pl
from jax.experimental.pallas import tpu as pltpu
from jax.experimental.pallas import tpu_sc as plsc

N = 10000
E = 320000
D = 128
NB = 16
NC = 2
NS = 16
NW = NC * NS
L = 16

BE = 2560
C2 = 400
C4 = 1280
FPT = 4

_NEG_INF = float("-inf")


def _proj_body(x_ref, wex_ref, wnx_ref, q_ref, pT_ref):
    xb = x_ref[...]
    q_ref[...] = lax.dot_general(xb, wex_ref[...], (((1,), (1,)), ((), ())),
                                 preferred_element_type=jnp.float32)
    pT_ref[...] = lax.dot_general(wnx_ref[...], xb, (((1,), (1,)), ((), ())),
                                  preferred_element_type=jnp.float32)


def _k1a(x, Wex, Wnx):
    return pl.pallas_call(
        _proj_body,
        out_shape=[
            jax.ShapeDtypeStruct((N, D), jnp.float32),
            jax.ShapeDtypeStruct((D, N), jnp.float32),
        ],
    )(x, Wex, Wnx)


def _r_body(ea_ref, wea_ref, be_ref, r_ref):
    r_ref[...] = lax.dot_general(ea_ref[...], wea_ref[...],
                                 (((1,), (1,)), ((), ())),
                                 preferred_element_type=jnp.float32) + be_ref[...]


def _k1b(edge_attr, Wea, be2d):
    return pl.pallas_call(
        _r_body,
        grid=(E // BE,),
        in_specs=[
            pl.BlockSpec((BE, 16), lambda i: (i, 0)),
            pl.BlockSpec((D, 16), lambda i: (0, 0)),
            pl.BlockSpec((1, D), lambda i: (0, 0)),
        ],
        out_specs=pl.BlockSpec((BE, D), lambda i: (i, 0)),
        out_shape=jax.ShapeDtypeStruct((E, D), jnp.float32),
    )(edge_attr, Wea, be2d)


def _hT_body(e_ref, wne_ref, bn_ref, hT_ref):
    hT_ref[...] = lax.dot_general(wne_ref[...], e_ref[...],
                                  (((1,), (1,)), ((), ())),
                                  preferred_element_type=jnp.float32) + bn_ref[...]


def _k3(e, Wne, bn2d):
    return pl.pallas_call(
        _hT_body,
        grid=(E // BE,),
        in_specs=[
            pl.BlockSpec((BE, D), lambda i: (i, 0)),
            pl.BlockSpec((D, D), lambda i: (0, 0)),
            pl.BlockSpec((D, 1), lambda i: (0, 0)),
        ],
        out_specs=pl.BlockSpec((D, BE), lambda i: (0, i)),
        out_shape=jax.ShapeDtypeStruct((D, E), jnp.float32),
    )(e, Wne, bn2d)


def _final_body(aggT_ref, b2d_ref, glob_ref, w2a_ref, b2_ref, w2g_ref,
                wgg_ref, wgm_ref, bg_ref, xn_ref, u_ref):
    aggT = aggT_ref[...]
    xb = lax.dot_general(aggT, w2a_ref[...], (((0,), (1,)), ((), ())),
                         preferred_element_type=jnp.float32)
    oh = (b2d_ref[...] == lax.broadcasted_iota(jnp.int32, (N, NB), 1))
    oh = oh.astype(jnp.float32)
    G2 = lax.dot_general(glob_ref[...], w2g_ref[...], (((1,), (1,)), ((), ())),
                         preferred_element_type=jnp.float32)
    xn = xb + lax.dot_general(oh, G2, (((1,), (0,)), ((), ())),
                              preferred_element_type=jnp.float32) + b2_ref[...]
    xn = jnp.maximum(xn, 0.0)
    xn_ref[...] = xn

    S = lax.dot_general(oh, xn, (((0,), (0,)), ((), ())),
                        preferred_element_type=jnp.float32)
    cnt = lax.dot_general(oh, jnp.ones((N, D), jnp.float32),
                          (((0,), (0,)), ((), ())),
                          preferred_element_type=jnp.float32)
    mean = S / jnp.maximum(cnt, 1.0)
    u = (lax.dot_general(glob_ref[...], wgg_ref[...],
                         (((1,), (1,)), ((), ())),
                         preferred_element_type=jnp.float32)
         + lax.dot_general(mean, wgm_ref[...], (((1,), (1,)), ((), ())),
                           preferred_element_type=jnp.float32)
         + bg_ref[...])
    u_ref[...] = jnp.maximum(u, 0.0)


def _k5(aggT, batch2d, glob, W2a, b22d, W2g, Wgg, Wgm, bg2d):
    return pl.pallas_call(
        _final_body,
        out_shape=[
            jax.ShapeDtypeStruct((N, D), jnp.float32),
            jax.ShapeDtypeStruct((NB, 32), jnp.float32),
        ],
    )(aggT, batch2d, glob, W2a, b22d, W2g, Wgg, Wgm, bg2d)


def _sc_mesh():
    return plsc.VectorSubcoreMesh(core_axis_name="c", subcore_axis_name="s")


_SC_PARAMS = pltpu.CompilerParams(needs_layout_passes=False,
                                  use_tc_tiling_on_sc=False)


def _edge_sc_body(q_hbm, row_hbm, r_hbm, e_hbm, idx_v, qrows_v, rbuf_v, sem):
    wid = lax.axis_index("s") * NC + lax.axis_index("c")
    base0 = wid * (E // NW)
    nchunks = (E // NW) // C2

    def chunk(ci, carry):
        base = base0 + ci * C2
        pltpu.sync_copy(row_hbm.at[pl.ds(base, C2)], idx_v)
        cp = pltpu.async_copy(q_hbm.at[idx_v], qrows_v, sem)
        pltpu.sync_copy(r_hbm.at[pl.ds(base, C2)], rbuf_v)
        cp.wait()

        def rowloop(i, c2):
            for j in range(D // L):
                sl = pl.ds(j * L, L)
                v = qrows_v[i, sl] + rbuf_v[i, sl]
                rbuf_v[i, sl] = jnp.maximum(v, 0.0)
            return c2

        lax.fori_loop(0, C2, rowloop, 0)
        pltpu.sync_copy(rbuf_v, e_hbm.at[pl.ds(base, C2)])
        return carry

    lax.fori_loop(0, nchunks, chunk, 0)


def _k2(q, row, r):
    fn = pl.kernel(
        _edge_sc_body,
        out_type=jax.ShapeDtypeStruct((E, D), jnp.float32),
        mesh=_sc_mesh(),
        compiler_params=_SC_PARAMS,
        scratch_types=[
            pltpu.VMEM((C2,), jnp.int32),
            pltpu.VMEM((C2, D), jnp.float32),
            pltpu.VMEM((C2, D), jnp.float32),
            pltpu.SemaphoreType.DMA,
        ],
    )
    return fn(q, row, r)


def _segmax_body(hT_hbm, row_hbm, col_hbm, pT_hbm, aggT_hbm,
                 p_v, agg_v, tmp_v,
                 hbuf0, rowb0, colb0, hbuf1, rowb1, colb1, sem0, sem1):
    wid = lax.axis_index("s") * NC + lax.axis_index("c")
    f0 = wid * FPT
    slots = ((hbuf0, rowb0, colb0, sem0), (hbuf1, rowb1, colb1, sem1))
    nchunks = E // C4

    def issue(ci, slot):
        hbuf, rowb, colb, sem = slots[slot]
        base = ci * C4
        pltpu.async_copy(row_hbm.at[pl.ds(base, C4)], rowb, sem)
        pltpu.async_copy(col_hbm.at[pl.ds(base, C4)], colb, sem)
        for f in range(FPT):
            pltpu.async_copy(hT_hbm.at[f0 + f, pl.ds(base, C4)],
                             hbuf.at[pl.ds(f * C4, C4)], sem)

    def wait(ci, slot):
        hbuf, rowb, colb, sem = slots[slot]
        base = ci * C4
        pltpu.make_async_copy(row_hbm.at[pl.ds(base, C4)], rowb, sem).wait()
        pltpu.make_async_copy(col_hbm.at[pl.ds(base, C4)], colb, sem).wait()
        for f in range(FPT):
            pltpu.make_async_copy(hT_hbm.at[f0 + f, pl.ds(base, C4)],
                                  hbuf.at[pl.ds(f * C4, C4)], sem).wait()

    issue(0, 0)
    issue(1, 1)

    for f in range(FPT):
        pltpu.sync_copy(pT_hbm.at[f0 + f], p_v.at[pl.ds(f * N, N)])

    neg = jnp.full((L,), _NEG_INF, jnp.float32)

    def initloop(i, carry):
        agg_v[pl.ds(i * L, L)] = neg
        return carry

    lax.fori_loop(0, (FPT * N) // L, initloop, 0)

    ids = lax.broadcasted_iota(jnp.int32, (L,), 0)

    def process(hbuf, rowb, colb):
        def group(k):
            sl = pl.ds(k * L, L)
            rv = rowb[sl]
            cv = colb[sl]
            plsc.store_scatter(tmp_v, [rv], ids)
            got = plsc.load_gather(tmp_v, [rv])
            has_dup = jnp.any(got != ids)
            hvs = []
            for f in range(FPT):
                hvs.append(hbuf[pl.ds(f * C4 + k * L, L)]
                           + plsc.load_gather(p_v, [cv + (f * N)]))

            def fast(_):
                for f in range(FPT):
                    ri = rv + (f * N)
                    cur = plsc.load_gather(agg_v, [ri])
                    plsc.store_scatter(agg_v, [ri], jnp.maximum(cur, hvs[f]))
                return 0

            def slow(_):
                for f in range(FPT):
                    ri = rv + (f * N)
                    hv = hvs[f]

                    def cond(m):
                        return jnp.any(m)

                    def body(m):
                        cur = plsc.load_gather(agg_v, [ri])
                        val = jnp.maximum(cur, hv)
                        plsc.store_scatter(agg_v, [ri], val, mask=m)
                        chk = plsc.load_gather(agg_v, [ri])
                        return m & (chk < hv)

                    lax.while_loop(cond, body, jnp.ones((L,), jnp.bool_))
                return 0

            lax.cond(has_dup, slow, fast, 0)

        def group2(g, c2):
            group(2 * g)
            group(2 * g + 1)
            return c2

        lax.fori_loop(0, C4 // L // 2, group2, 0)

    def pair(i, carry):
        for slot in (0, 1):
            ci = 2 * i + slot
            wait(ci, slot)
            hbuf, rowb, colb, _ = slots[slot]
            process(hbuf, rowb, colb)

            @pl.when(ci + 2 < nchunks)
            def _():
                issue(ci + 2, slot)
        return carry

    lax.fori_loop(0, nchunks // 2, pair, 0)

    def fixloop(i, carry):
        sl = pl.ds(i * L, L)
        v = agg_v[sl]
        ok = (v - v) == 0.0
        agg_v[sl] = jnp.where(ok, v, 0.0)
        return carry

    lax.fori_loop(0, (FPT * N) // L, fixloop, 0)
    for f in range(FPT):
        pltpu.sync_copy(agg_v.at[pl.ds(f * N, N)], aggT_hbm.at[f0 + f])


def _k4(hT, row, col, pT):
    fn = pl.kernel(
        _segmax_body,
        out_type=jax.ShapeDtypeStruct((D, N), jnp.float32),
        mesh=_sc_mesh(),
        compiler_params=_SC_PARAMS,
        scratch_types=[
            pltpu.VMEM((FPT * N,), jnp.float32),
            pltpu.VMEM((FPT * N,), jnp.float32),
            pltpu.VMEM((N,), jnp.int32),
            pltpu.VMEM((FPT * C4,), jnp.float32),
            pltpu.VMEM((C4,), jnp.int32),
            pltpu.VMEM((C4,), jnp.int32),
            pltpu.VMEM((FPT * C4,), jnp.float32),
            pltpu.VMEM((C4,), jnp.int32),
            pltpu.VMEM((C4,), jnp.int32),
            pltpu.SemaphoreType.DMA,
            pltpu.SemaphoreType.DMA,
        ],
    )
    return fn(hT, row, col, pT)


def kernel(x, edge_index, edge_attr, glob, batch,
           W_edge, b_edge, W_node, b_node, W_node2, b_node2, W_glob, b_glob):
    row = edge_index[0]
    col = edge_index[1]
    Wex, Wea = W_edge[:, :D], W_edge[:, D:]
    Wnx, Wne = W_node[:, :D], W_node[:, D:]
    W2a, W2g = W_node2[:, :D], W_node2[:, D:]
    Wgg, Wgm = W_glob[:, :32], W_glob[:, 32:]
    be2d = b_edge.reshape(1, D)
    bn2d = b_node.reshape(D, 1)
    b22d = b_node2.reshape(1, D)
    bg2d = b_glob.reshape(1, 32)
    batch2d = batch.reshape(N, 1)

    q, pT = _k1a(x, Wex, Wnx)
    r = _k1b(edge_attr, Wea, be2d)
    e = _k2(q, row, r)
    hT = _k3(e, Wne, bn2d)
    aggT = _k4(hT, row, col, pT)
    xn, u = _k5(aggT, batch2d, glob, W2a, b22d, W2g, Wgg, Wgm, bg2d)
    return (xn, e, u)


# speedup vs baseline: 3.4428x; 1.1187x over previous
import functools
import jax
import jax.numpy as jnp
from jax import lax
from jax.experimental import pallas as pl
from jax.experimental.pallas import tpu as pltpu
from jax.experimental.pallas import tpu_sc as plsc

N = 10000
E = 320000
D = 128
NB = 16
NC = 2
NS = 16
NW = NC * NS
L = 16

BE = 2560
C2 = 400
C4 = 1280
FPT = 4

_NEG_INF = float("-inf")


def _proj_body(x_ref, wex_ref, wnx_ref, q_ref, pT_ref):
    xb = x_ref[...]
    q_ref[...] = lax.dot_general(xb, wex_ref[...], (((1,), (1,)), ((), ())),
                                 preferred_element_type=jnp.float32)
    pT_ref[...] = lax.dot_general(wnx_ref[...], xb, (((1,), (1,)), ((), ())),
                                  preferred_element_type=jnp.float32)


def _k1a(x, Wex, Wnx):
    return pl.pallas_call(
        _proj_body,
        out_shape=[
            jax.ShapeDtypeStruct((N, D), jnp.float32),
            jax.ShapeDtypeStruct((D, N), jnp.float32),
        ],
    )(x, Wex, Wnx)


def _r_body(ea_ref, wea_ref, be_ref, r_ref):
    r_ref[...] = lax.dot_general(ea_ref[...], wea_ref[...],
                                 (((1,), (1,)), ((), ())),
                                 preferred_element_type=jnp.float32) + be_ref[...]


def _k1b(edge_attr, Wea, be2d):
    return pl.pallas_call(
        _r_body,
        grid=(E // BE,),
        in_specs=[
            pl.BlockSpec((BE, 16), lambda i: (i, 0)),
            pl.BlockSpec((D, 16), lambda i: (0, 0)),
            pl.BlockSpec((1, D), lambda i: (0, 0)),
        ],
        out_specs=pl.BlockSpec((BE, D), lambda i: (i, 0)),
        out_shape=jax.ShapeDtypeStruct((E, D), jnp.float32),
    )(edge_attr, Wea, be2d)


def _hT_body(e_ref, wne_ref, bn_ref, hT_ref):
    hT_ref[...] = lax.dot_general(wne_ref[...], e_ref[...],
                                  (((1,), (1,)), ((), ())),
                                  preferred_element_type=jnp.float32) + bn_ref[...]


def _k3(e, Wne, bn2d):
    return pl.pallas_call(
        _hT_body,
        grid=(E // BE,),
        in_specs=[
            pl.BlockSpec((BE, D), lambda i: (i, 0)),
            pl.BlockSpec((D, D), lambda i: (0, 0)),
            pl.BlockSpec((D, 1), lambda i: (0, 0)),
        ],
        out_specs=pl.BlockSpec((D, BE), lambda i: (0, i)),
        out_shape=jax.ShapeDtypeStruct((D, E), jnp.float32),
    )(e, Wne, bn2d)


def _final_body(aggT_ref, b2d_ref, glob_ref, w2a_ref, b2_ref, w2g_ref,
                wgg_ref, wgm_ref, bg_ref, xn_ref, u_ref):
    aggT = aggT_ref[...]
    xb = lax.dot_general(aggT, w2a_ref[...], (((0,), (1,)), ((), ())),
                         preferred_element_type=jnp.float32)
    oh = (b2d_ref[...] == lax.broadcasted_iota(jnp.int32, (N, NB), 1))
    oh = oh.astype(jnp.float32)
    G2 = lax.dot_general(glob_ref[...], w2g_ref[...], (((1,), (1,)), ((), ())),
                         preferred_element_type=jnp.float32)
    xn = xb + lax.dot_general(oh, G2, (((1,), (0,)), ((), ())),
                              preferred_element_type=jnp.float32) + b2_ref[...]
    xn = jnp.maximum(xn, 0.0)
    xn_ref[...] = xn

    S = lax.dot_general(oh, xn, (((0,), (0,)), ((), ())),
                        preferred_element_type=jnp.float32)
    cnt = lax.dot_general(oh, jnp.ones((N, D), jnp.float32),
                          (((0,), (0,)), ((), ())),
                          preferred_element_type=jnp.float32)
    mean = S / jnp.maximum(cnt, 1.0)
    u = (lax.dot_general(glob_ref[...], wgg_ref[...],
                         (((1,), (1,)), ((), ())),
                         preferred_element_type=jnp.float32)
         + lax.dot_general(mean, wgm_ref[...], (((1,), (1,)), ((), ())),
                           preferred_element_type=jnp.float32)
         + bg_ref[...])
    u_ref[...] = jnp.maximum(u, 0.0)


def _k5(aggT, batch2d, glob, W2a, b22d, W2g, Wgg, Wgm, bg2d):
    return pl.pallas_call(
        _final_body,
        out_shape=[
            jax.ShapeDtypeStruct((N, D), jnp.float32),
            jax.ShapeDtypeStruct((NB, 32), jnp.float32),
        ],
    )(aggT, batch2d, glob, W2a, b22d, W2g, Wgg, Wgm, bg2d)


def _sc_mesh():
    return plsc.VectorSubcoreMesh(core_axis_name="c", subcore_axis_name="s")


_SC_PARAMS = pltpu.CompilerParams(needs_layout_passes=False,
                                  use_tc_tiling_on_sc=False)


def _edge_sc_body(q_hbm, row_hbm, r_hbm, e_hbm, idx_v, qrows_v, rbuf_v, sem):
    wid = lax.axis_index("s") * NC + lax.axis_index("c")
    base0 = wid * (E // NW)
    nchunks = (E // NW) // C2

    def chunk(ci, carry):
        base = base0 + ci * C2
        pltpu.sync_copy(row_hbm.at[pl.ds(base, C2)], idx_v)
        cp = pltpu.async_copy(q_hbm.at[idx_v], qrows_v, sem)
        pltpu.sync_copy(r_hbm.at[pl.ds(base, C2)], rbuf_v)
        cp.wait()

        def rowloop(i, c2):
            for j in range(D // L):
                sl = pl.ds(j * L, L)
                v = qrows_v[i, sl] + rbuf_v[i, sl]
                rbuf_v[i, sl] = jnp.maximum(v, 0.0)
            return c2

        lax.fori_loop(0, C2, rowloop, 0)
        pltpu.sync_copy(rbuf_v, e_hbm.at[pl.ds(base, C2)])
        return carry

    lax.fori_loop(0, nchunks, chunk, 0)


def _k2(q, row, r):
    fn = pl.kernel(
        _edge_sc_body,
        out_type=jax.ShapeDtypeStruct((E, D), jnp.float32),
        mesh=_sc_mesh(),
        compiler_params=_SC_PARAMS,
        scratch_types=[
            pltpu.VMEM((C2,), jnp.int32),
            pltpu.VMEM((C2, D), jnp.float32),
            pltpu.VMEM((C2, D), jnp.float32),
            pltpu.SemaphoreType.DMA,
        ],
    )
    return fn(q, row, r)


def _segmax_body(hT_hbm, row_hbm, col_hbm, pT_hbm, aggT_hbm,
                 p_v, agg_v, tmp_v,
                 hbuf0, rowb0, colb0, hbuf1, rowb1, colb1, sem0, sem1):
    wid = lax.axis_index("s") * NC + lax.axis_index("c")
    f0 = wid * FPT
    slots = ((hbuf0, rowb0, colb0, sem0), (hbuf1, rowb1, colb1, sem1))
    nchunks = E // C4

    def issue(ci, slot):
        hbuf, rowb, colb, sem = slots[slot]
        base = ci * C4
        pltpu.async_copy(row_hbm.at[pl.ds(base, C4)], rowb, sem)
        pltpu.async_copy(col_hbm.at[pl.ds(base, C4)], colb, sem)
        for f in range(FPT):
            pltpu.async_copy(hT_hbm.at[f0 + f, pl.ds(base, C4)],
                             hbuf.at[pl.ds(f * C4, C4)], sem)

    def wait(ci, slot):
        hbuf, rowb, colb, sem = slots[slot]
        base = ci * C4
        pltpu.make_async_copy(row_hbm.at[pl.ds(base, C4)], rowb, sem).wait()
        pltpu.make_async_copy(col_hbm.at[pl.ds(base, C4)], colb, sem).wait()
        for f in range(FPT):
            pltpu.make_async_copy(hT_hbm.at[f0 + f, pl.ds(base, C4)],
                                  hbuf.at[pl.ds(f * C4, C4)], sem).wait()

    issue(0, 0)
    issue(1, 1)

    for f in range(FPT):
        pltpu.sync_copy(pT_hbm.at[f0 + f], p_v.at[pl.ds(f * N, N)])

    neg = jnp.full((L,), _NEG_INF, jnp.float32)

    def initloop(i, carry):
        agg_v[pl.ds(i * L, L)] = neg
        return carry

    lax.fori_loop(0, (FPT * N) // L, initloop, 0)

    ids = lax.broadcasted_iota(jnp.int32, (L,), 0)

    def process(hbuf, rowb, colb):
        def load_group(k):
            sl = pl.ds(k * L, L)
            rv = rowb[sl]
            cv = colb[sl]
            plsc.store_scatter(tmp_v, [rv], ids)
            got = plsc.load_gather(tmp_v, [rv])
            bad = got != ids
            hvs = []
            for f in range(FPT):
                hvs.append(hbuf[pl.ds(f * C4 + k * L, L)]
                           + plsc.load_gather(p_v, [cv + (f * N)]))
            return rv, bad, hvs

        def fast_group(rv, hvs):
            for f in range(FPT):
                ri = rv + (f * N)
                cur = plsc.load_gather(agg_v, [ri])
                plsc.store_scatter(agg_v, [ri], jnp.maximum(cur, hvs[f]))

        def slow_group(rv, hvs):
            for f in range(FPT):
                ri = rv + (f * N)
                hv = hvs[f]

                def cond(m):
                    return jnp.any(m)

                def body(m):
                    cur = plsc.load_gather(agg_v, [ri])
                    val = jnp.maximum(cur, hv)
                    plsc.store_scatter(agg_v, [ri], val, mask=m)
                    chk = plsc.load_gather(agg_v, [ri])
                    return m & (chk < hv)

                lax.while_loop(cond, body, jnp.ones((L,), jnp.bool_))

        def group2(g, c2):
            rv0, bad0, hvs0 = load_group(2 * g)
            rv1, bad1, hvs1 = load_group(2 * g + 1)
            has_dup = jnp.any(bad0 | bad1)

            def fast(_):
                fast_group(rv0, hvs0)
                fast_group(rv1, hvs1)
                return 0

            def slow(_):
                slow_group(rv0, hvs0)
                slow_group(rv1, hvs1)
                return 0

            lax.cond(has_dup, slow, fast, 0)
            return c2

        lax.fori_loop(0, C4 // L // 2, group2, 0)

    def pair(i, carry):
        for slot in (0, 1):
            ci = 2 * i + slot
            wait(ci, slot)
            hbuf, rowb, colb, _ = slots[slot]
            process(hbuf, rowb, colb)

            @pl.when(ci + 2 < nchunks)
            def _():
                issue(ci + 2, slot)
        return carry

    lax.fori_loop(0, nchunks // 2, pair, 0)

    def fixloop(i, carry):
        sl = pl.ds(i * L, L)
        v = agg_v[sl]
        ok = (v - v) == 0.0
        agg_v[sl] = jnp.where(ok, v, 0.0)
        return carry

    lax.fori_loop(0, (FPT * N) // L, fixloop, 0)
    for f in range(FPT):
        pltpu.sync_copy(agg_v.at[pl.ds(f * N, N)], aggT_hbm.at[f0 + f])


def _k4(hT, row, col, pT):
    fn = pl.kernel(
        _segmax_body,
        out_type=jax.ShapeDtypeStruct((D, N), jnp.float32),
        mesh=_sc_mesh(),
        compiler_params=_SC_PARAMS,
        scratch_types=[
            pltpu.VMEM((FPT * N,), jnp.float32),
            pltpu.VMEM((FPT * N,), jnp.float32),
            pltpu.VMEM((N,), jnp.int32),
            pltpu.VMEM((FPT * C4,), jnp.float32),
            pltpu.VMEM((C4,), jnp.int32),
            pltpu.VMEM((C4,), jnp.int32),
            pltpu.VMEM((FPT * C4,), jnp.float32),
            pltpu.VMEM((C4,), jnp.int32),
            pltpu.VMEM((C4,), jnp.int32),
            pltpu.SemaphoreType.DMA,
            pltpu.SemaphoreType.DMA,
        ],
    )
    return fn(hT, row, col, pT)


def kernel(x, edge_index, edge_attr, glob, batch,
           W_edge, b_edge, W_node, b_node, W_node2, b_node2, W_glob, b_glob):
    row = edge_index[0]
    col = edge_index[1]
    Wex, Wea = W_edge[:, :D], W_edge[:, D:]
    Wnx, Wne = W_node[:, :D], W_node[:, D:]
    W2a, W2g = W_node2[:, :D], W_node2[:, D:]
    Wgg, Wgm = W_glob[:, :32], W_glob[:, 32:]
    be2d = b_edge.reshape(1, D)
    bn2d = b_node.reshape(D, 1)
    b22d = b_node2.reshape(1, D)
    bg2d = b_glob.reshape(1, 32)
    batch2d = batch.reshape(N, 1)

    q, pT = _k1a(x, Wex, Wnx)
    r = _k1b(edge_attr, Wea, be2d)
    e = _k2(q, row, r)
    hT = _k3(e, Wne, bn2d)
    aggT = _k4(hT, row, col, pT)
    xn, u = _k5(aggT, batch2d, glob, W2a, b22d, W2g, Wgg, Wgm, bg2d)
    return (xn, e, u)


# speedup vs baseline: 3.5651x; 1.0355x over previous
import functools
import jax
import jax.numpy as jnp
from jax import lax
from jax.experimental import pallas as pl
from jax.experimental.pallas import tpu as pltpu
from jax.experimental.pallas import tpu_sc as plsc

N = 10000
E = 320000
D = 128
NB = 16
NC = 2
NS = 16
NW = NC * NS
L = 16

BE = 2560
C2 = 40
C4 = 640
FPT = 8
EH = 2
MP = 10000

_NEG_INF = float("-inf")


def _proj_body(x_ref, wex_ref, wnx_ref, q_ref, p_ref):
    xb = x_ref[...]
    q_ref[...] = lax.dot_general(xb, wex_ref[...], (((1,), (1,)), ((), ())),
                                 preferred_element_type=jnp.float32)
    p_ref[...] = lax.dot_general(xb, wnx_ref[...], (((1,), (1,)), ((), ())),
                                 preferred_element_type=jnp.float32)


def _k1a(x, Wex, Wnx):
    return pl.pallas_call(
        _proj_body,
        out_shape=[
            jax.ShapeDtypeStruct((N, D), jnp.float32),
            jax.ShapeDtypeStruct((N, D), jnp.float32),
        ],
    )(x, Wex, Wnx)


def _r_body(ea_ref, wea_ref, be_ref, r_ref):
    r_ref[...] = lax.dot_general(ea_ref[...], wea_ref[...],
                                 (((1,), (1,)), ((), ())),
                                 preferred_element_type=jnp.float32) + be_ref[...]


def _k1b(edge_attr, Wea, be2d):
    return pl.pallas_call(
        _r_body,
        grid=(E // BE,),
        in_specs=[
            pl.BlockSpec((BE, 16), lambda i: (i, 0)),
            pl.BlockSpec((D, 16), lambda i: (0, 0)),
            pl.BlockSpec((1, D), lambda i: (0, 0)),
        ],
        out_specs=pl.BlockSpec((BE, D), lambda i: (i, 0)),
        out_shape=jax.ShapeDtypeStruct((E, D), jnp.float32),
    )(edge_attr, Wea, be2d)


def _hT_body(e_ref, pc_ref, wne_ref, bn_ref, hT_ref):
    hT_ref[...] = (lax.dot_general(wne_ref[...], e_ref[...],
                                   (((1,), (1,)), ((), ())),
                                   preferred_element_type=jnp.float32)
                   + lax.transpose(pc_ref[...], (1, 0)) + bn_ref[...])


def _k3(e, pc, Wne, bn2d):
    return pl.pallas_call(
        _hT_body,
        grid=(E // BE,),
        in_specs=[
            pl.BlockSpec((BE, D), lambda i: (i, 0)),
            pl.BlockSpec((BE, D), lambda i: (i, 0)),
            pl.BlockSpec((D, D), lambda i: (0, 0)),
            pl.BlockSpec((D, 1), lambda i: (0, 0)),
        ],
        out_specs=pl.BlockSpec((D, BE), lambda i: (0, i)),
        out_shape=jax.ShapeDtypeStruct((D, E), jnp.float32),
    )(e, pc, Wne, bn2d)


def _final_body(aggT_ref, b2d_ref, glob_ref, w2a_ref, b2_ref, w2g_ref,
                wgg_ref, wgm_ref, bg_ref, xn_ref, u_ref):
    aggT = aggT_ref[...]
    xb = lax.dot_general(aggT, w2a_ref[...], (((0,), (1,)), ((), ())),
                         preferred_element_type=jnp.float32)
    oh = (b2d_ref[...] == lax.broadcasted_iota(jnp.int32, (N, NB), 1))
    oh = oh.astype(jnp.float32)
    G2 = lax.dot_general(glob_ref[...], w2g_ref[...], (((1,), (1,)), ((), ())),
                         preferred_element_type=jnp.float32)
    xn = xb + lax.dot_general(oh, G2, (((1,), (0,)), ((), ())),
                              preferred_element_type=jnp.float32) + b2_ref[...]
    xn = jnp.maximum(xn, 0.0)
    xn_ref[...] = xn

    S = lax.dot_general(oh, xn, (((0,), (0,)), ((), ())),
                        preferred_element_type=jnp.float32)
    cnt = lax.dot_general(oh, jnp.ones((N, D), jnp.float32),
                          (((0,), (0,)), ((), ())),
                          preferred_element_type=jnp.float32)
    mean = S / jnp.maximum(cnt, 1.0)
    u = (lax.dot_general(glob_ref[...], wgg_ref[...],
                         (((1,), (1,)), ((), ())),
                         preferred_element_type=jnp.float32)
         + lax.dot_general(mean, wgm_ref[...], (((1,), (1,)), ((), ())),
                           preferred_element_type=jnp.float32)
         + bg_ref[...])
    u_ref[...] = jnp.maximum(u, 0.0)


def _k5(aggT, batch2d, glob, W2a, b22d, W2g, Wgg, Wgm, bg2d):
    return pl.pallas_call(
        _final_body,
        out_shape=[
            jax.ShapeDtypeStruct((N, D), jnp.float32),
            jax.ShapeDtypeStruct((NB, 32), jnp.float32),
        ],
    )(aggT, batch2d, glob, W2a, b22d, W2g, Wgg, Wgm, bg2d)


def _sc_mesh():
    return plsc.VectorSubcoreMesh(core_axis_name="c", subcore_axis_name="s")


_SC_PARAMS = pltpu.CompilerParams(needs_layout_passes=False,
                                  use_tc_tiling_on_sc=False)


def _edge_sc_body(q_hbm, p_hbm, row_hbm, col_hbm, r_hbm, e_hbm, pc_hbm,
                  idxr0, idxc0, qrows0, prows0, rbuf0,
                  idxr1, idxc1, qrows1, prows1, rbuf1,
                  semi0, semg0, semo0, semi1, semg1, semo1):
    wid = lax.axis_index("s") * NC + lax.axis_index("c")
    base0 = wid * (E // NW)
    nchunks = (E // NW) // C2
    slots = ((idxr0, idxc0, qrows0, prows0, rbuf0, semi0, semg0, semo0),
             (idxr1, idxc1, qrows1, prows1, rbuf1, semi1, semg1, semo1))

    def issue_idx(ci, s):
        idxr, idxc, _, _, _, semi, _, _ = slots[s]
        base = base0 + ci * C2
        pltpu.async_copy(row_hbm.at[pl.ds(base, C2)], idxr, semi)
        pltpu.async_copy(col_hbm.at[pl.ds(base, C2)], idxc, semi)

    def wait_idx(ci, s):
        idxr, idxc, _, _, _, semi, _, _ = slots[s]
        base = base0 + ci * C2
        pltpu.make_async_copy(row_hbm.at[pl.ds(base, C2)], idxr, semi).wait()
        pltpu.make_async_copy(col_hbm.at[pl.ds(base, C2)], idxc, semi).wait()

    def issue_g(ci, s):
        idxr, idxc, qrows, prows, rbuf, _, semg, _ = slots[s]
        base = base0 + ci * C2
        pltpu.async_copy(q_hbm.at[idxr], qrows, semg)
        pltpu.async_copy(p_hbm.at[idxc], prows, semg)
        pltpu.async_copy(r_hbm.at[pl.ds(base, C2)], rbuf, semg)

    def wait_g(ci, s):
        idxr, idxc, qrows, prows, rbuf, _, semg, _ = slots[s]
        base = base0 + ci * C2
        pltpu.make_async_copy(q_hbm.at[idxr], qrows, semg).wait()
        pltpu.make_async_copy(p_hbm.at[idxc], prows, semg).wait()
        pltpu.make_async_copy(r_hbm.at[pl.ds(base, C2)], rbuf, semg).wait()

    def issue_out(ci, s):
        _, _, qrows, prows, _, _, _, semo = slots[s]
        base = base0 + ci * C2
        pltpu.async_copy(qrows, e_hbm.at[pl.ds(base, C2)], semo)
        pltpu.async_copy(prows, pc_hbm.at[pl.ds(base, C2)], semo)

    def wait_out(ci, s):
        _, _, qrows, prows, _, _, _, semo = slots[s]
        base = base0 + ci * C2
        pltpu.make_async_copy(qrows, e_hbm.at[pl.ds(base, C2)], semo).wait()
        pltpu.make_async_copy(prows, pc_hbm.at[pl.ds(base, C2)], semo).wait()

    issue_idx(0, 0)
    issue_idx(1, 1)
    wait_idx(0, 0)
    issue_g(0, 0)

    def pair(i, carry):
        for s in (0, 1):
            ci = 2 * i + s
            so = 1 - s

            @pl.when((ci + 1 < nchunks) & (ci >= 1))
            def _():
                wait_out(ci - 1, so)

            @pl.when(ci + 1 < nchunks)
            def _():
                wait_idx(ci + 1, so)
                issue_g(ci + 1, so)

            wait_g(ci, s)
            _, _, qrows, prows, rbuf, _, _, _ = slots[s]

            def rowloop(ii, c2):
                for j in range(D // L):
                    slc = pl.ds(j * L, L)
                    v = qrows[ii, slc] + rbuf[ii, slc]
                    qrows[ii, slc] = jnp.maximum(v, 0.0)
                return c2

            lax.fori_loop(0, C2, rowloop, 0)
            issue_out(ci, s)

            @pl.when(ci + 2 < nchunks)
            def _():
                issue_idx(ci + 2, s)
        return carry

    lax.fori_loop(0, nchunks // 2, pair, 0)
    wait_out(nchunks - 2, 0)
    wait_out(nchunks - 1, 1)


def _k2(q, p, row, col, r):
    fn = pl.kernel(
        _edge_sc_body,
        out_type=[
            jax.ShapeDtypeStruct((E, D), jnp.float32),
            jax.ShapeDtypeStruct((E, D), jnp.float32),
        ],
        mesh=_sc_mesh(),
        compiler_params=_SC_PARAMS,
        scratch_types=[
            pltpu.VMEM((C2,), jnp.int32),
            pltpu.VMEM((C2,), jnp.int32),
            pltpu.VMEM((C2, D), jnp.float32),
            pltpu.VMEM((C2, D), jnp.float32),
            pltpu.VMEM((C2, D), jnp.float32),
            pltpu.VMEM((C2,), jnp.int32),
            pltpu.VMEM((C2,), jnp.int32),
            pltpu.VMEM((C2, D), jnp.float32),
            pltpu.VMEM((C2, D), jnp.float32),
            pltpu.VMEM((C2, D), jnp.float32),
            pltpu.SemaphoreType.DMA,
            pltpu.SemaphoreType.DMA,
            pltpu.SemaphoreType.DMA,
            pltpu.SemaphoreType.DMA,
            pltpu.SemaphoreType.DMA,
            pltpu.SemaphoreType.DMA,
        ],
    )
    return fn(q, p, row, col, r)


def _segmax_body(hT_hbm, row_hbm, aggT_hbm, agg_v, tmp_v,
                 hbuf0, rowb0, hbuf1, rowb1, mbuf, shr, sem0, sem1):
    c = lax.axis_index("c")
    s = lax.axis_index("s")
    pid = s // 2
    eh = s % 2
    f0 = (c * 8 + pid) * FPT
    ebase = eh * (E // EH)
    nchunks = (E // EH) // C4
    slots = ((hbuf0, rowb0, sem0), (hbuf1, rowb1, sem1))

    def issue(ci, slot):
        hbuf, rowb, sem = slots[slot]
        base = ebase + ci * C4
        pltpu.async_copy(row_hbm.at[pl.ds(base, C4)], rowb, sem)
        for f in range(FPT):
            pltpu.async_copy(hT_hbm.at[f0 + f, pl.ds(base, C4)],
                             hbuf.at[pl.ds(f * C4, C4)], sem)

    def wait(ci, slot):
        hbuf, rowb, sem = slots[slot]
        base = ebase + ci * C4
        pltpu.make_async_copy(row_hbm.at[pl.ds(base, C4)], rowb, sem).wait()
        for f in range(FPT):
            pltpu.make_async_copy(hT_hbm.at[f0 + f, pl.ds(base, C4)],
                                  hbuf.at[pl.ds(f * C4, C4)], sem).wait()

    issue(0, 0)
    issue(1, 1)

    neg = jnp.full((L,), _NEG_INF, jnp.float32)

    def initloop(i, carry):
        agg_v[pl.ds(i * L, L)] = neg
        return carry

    lax.fori_loop(0, (FPT * N) // L, initloop, 0)

    ids = lax.broadcasted_iota(jnp.int32, (L,), 0)

    def process(hbuf, rowb):
        def load_group(k):
            sl = pl.ds(k * L, L)
            rv = rowb[sl]
            plsc.store_scatter(tmp_v, [rv], ids)
            got = plsc.load_gather(tmp_v, [rv])
            bad = got != ids
            hvs = []
            for f in range(FPT):
                hvs.append(hbuf[pl.ds(f * C4 + k * L, L)])
            return rv, bad, hvs

        def fast_group(rv, hvs):
            for f in range(FPT):
                ri = rv + (f * N)
                cur = plsc.load_gather(agg_v, [ri])
                plsc.store_scatter(agg_v, [ri], jnp.maximum(cur, hvs[f]))

        def slow_group(rv, hvs):
            for f in range(FPT):
                ri = rv + (f * N)
                hv = hvs[f]

                def cond(m):
                    return jnp.any(m)

                def body(m):
                    cur = plsc.load_gather(agg_v, [ri])
                    val = jnp.maximum(cur, hv)
                    plsc.store_scatter(agg_v, [ri], val, mask=m)
                    chk = plsc.load_gather(agg_v, [ri])
                    return m & (chk < hv)

                lax.while_loop(cond, body, jnp.ones((L,), jnp.bool_))

        def group2(g, c2):
            rv0, bad0, hvs0 = load_group(2 * g)
            rv1, bad1, hvs1 = load_group(2 * g + 1)
            has_dup = jnp.any(bad0 | bad1)

            def fast(_):
                fast_group(rv0, hvs0)
                fast_group(rv1, hvs1)
                return 0

            def slow(_):
                slow_group(rv0, hvs0)
                slow_group(rv1, hvs1)
                return 0

            lax.cond(has_dup, slow, fast, 0)
            return c2

        lax.fori_loop(0, C4 // L // 2, group2, 0)

    def pair(i, carry):
        for slot in (0, 1):
            ci = 2 * i + slot
            wait(ci, slot)
            hbuf, rowb, _ = slots[slot]
            process(hbuf, rowb)

            @pl.when(ci + 2 < nchunks)
            def _():
                issue(ci + 2, slot)
        return carry

    lax.fori_loop(0, nchunks // 2, pair, 0)

    def mergeloop(j, carry):
        @pl.when(eh == 1)
        def _():
            pltpu.sync_copy(agg_v.at[pl.ds(j * MP, MP)], shr.at[pid])

        plsc.subcore_barrier()

        @pl.when(eh == 0)
        def _():
            pltpu.sync_copy(shr.at[pid], mbuf)

            def vloop(i, c2):
                slc = pl.ds(j * MP + i * L, L)
                agg_v[slc] = jnp.maximum(agg_v[slc], mbuf[pl.ds(i * L, L)])
                return c2

            lax.fori_loop(0, MP // L, vloop, 0)

        plsc.subcore_barrier()
        return carry

    lax.fori_loop(0, (FPT * N) // MP, mergeloop, 0)

    @pl.when(eh == 0)
    def _():
        def fixloop(i, carry):
            sl = pl.ds(i * L, L)
            v = agg_v[sl]
            ok = (v - v) == 0.0
            agg_v[sl] = jnp.where(ok, v, 0.0)
            return carry

        lax.fori_loop(0, (FPT * N) // L, fixloop, 0)
        for f in range(FPT):
            pltpu.sync_copy(agg_v.at[pl.ds(f * N, N)], aggT_hbm.at[f0 + f])


def _k4(hT, row):
    fn = pl.kernel(
        _segmax_body,
        out_type=jax.ShapeDtypeStruct((D, N), jnp.float32),
        mesh=_sc_mesh(),
        compiler_params=_SC_PARAMS,
        scratch_types=[
            pltpu.VMEM((FPT * N,), jnp.float32),
            pltpu.VMEM((N,), jnp.int32),
            pltpu.VMEM((FPT * C4,), jnp.float32),
            pltpu.VMEM((C4,), jnp.int32),
            pltpu.VMEM((FPT * C4,), jnp.float32),
            pltpu.VMEM((C4,), jnp.int32),
            pltpu.VMEM((MP,), jnp.float32),
            pltpu.VMEM_SHARED((8, MP), jnp.float32),
            pltpu.SemaphoreType.DMA,
            pltpu.SemaphoreType.DMA,
        ],
    )
    return fn(hT, row)


def kernel(x, edge_index, edge_attr, glob, batch,
           W_edge, b_edge, W_node, b_node, W_node2, b_node2, W_glob, b_glob):
    row = edge_index[0]
    col = edge_index[1]
    Wex, Wea = W_edge[:, :D], W_edge[:, D:]
    Wnx, Wne = W_node[:, :D], W_node[:, D:]
    W2a, W2g = W_node2[:, :D], W_node2[:, D:]
    Wgg, Wgm = W_glob[:, :32], W_glob[:, 32:]
    be2d = b_edge.reshape(1, D)
    bn2d = b_node.reshape(D, 1)
    b22d = b_node2.reshape(1, D)
    bg2d = b_glob.reshape(1, 32)
    batch2d = batch.reshape(N, 1)

    q, p = _k1a(x, Wex, Wnx)
    r = _k1b(edge_attr, Wea, be2d)
    e, pc = _k2(q, p, row, col, r)
    hT = _k3(e, pc, Wne, bn2d)
    aggT = _k4(hT, row)
    xn, u = _k5(aggT, batch2d, glob, W2a, b22d, W2g, Wgg, Wgm, bg2d)
    return (xn, e, u)


# speedup vs baseline: 4.2127x; 1.1816x over previous
import functools
import jax
import jax.numpy as jnp
from jax import lax
from jax.experimental import pallas as pl
from jax.experimental.pallas import tpu as pltpu
from jax.experimental.pallas import tpu_sc as plsc

N = 10000
E = 320000
D = 128
NB = 16
NC = 2
NS = 16
NW = NC * NS
L = 16

BE = 2560
C2 = 40
C4 = 640
FPT = 8
EH = 2
MP = 10000

_NEG_INF = float("-inf")


def _proj_body(x_ref, wex_ref, wnx_ref, q_ref, p_ref):
    xb = x_ref[...]
    q_ref[...] = lax.dot_general(xb, wex_ref[...], (((1,), (1,)), ((), ())),
                                 preferred_element_type=jnp.float32)
    p_ref[...] = lax.dot_general(xb, wnx_ref[...], (((1,), (1,)), ((), ())),
                                 preferred_element_type=jnp.float32)


def _k1a(x, Wex, Wnx):
    return pl.pallas_call(
        _proj_body,
        out_shape=[
            jax.ShapeDtypeStruct((N, D), jnp.float32),
            jax.ShapeDtypeStruct((N, D), jnp.float32),
        ],
    )(x, Wex, Wnx)


def _r_body(ea_ref, wea_ref, be_ref, r_ref):
    r_ref[...] = lax.dot_general(ea_ref[...], wea_ref[...],
                                 (((1,), (1,)), ((), ())),
                                 preferred_element_type=jnp.float32) + be_ref[...]


def _k1b(edge_attr, Wea, be2d):
    return pl.pallas_call(
        _r_body,
        grid=(E // BE,),
        in_specs=[
            pl.BlockSpec((BE, 16), lambda i: (i, 0)),
            pl.BlockSpec((D, 16), lambda i: (0, 0)),
            pl.BlockSpec((1, D), lambda i: (0, 0)),
        ],
        out_specs=pl.BlockSpec((BE, D), lambda i: (i, 0)),
        out_shape=jax.ShapeDtypeStruct((E, D), jnp.float32),
    )(edge_attr, Wea, be2d)


def _hT_body(e_ref, pc_ref, wne_ref, bn_ref, hT_ref):
    hT_ref[...] = (lax.dot_general(wne_ref[...], e_ref[...],
                                   (((1,), (1,)), ((), ())),
                                   preferred_element_type=jnp.float32)
                   + lax.transpose(pc_ref[...], (1, 0)) + bn_ref[...])


def _k3(e, pc, Wne, bn2d):
    return pl.pallas_call(
        _hT_body,
        grid=(E // BE,),
        in_specs=[
            pl.BlockSpec((BE, D), lambda i: (i, 0)),
            pl.BlockSpec((BE, D), lambda i: (i, 0)),
            pl.BlockSpec((D, D), lambda i: (0, 0)),
            pl.BlockSpec((D, 1), lambda i: (0, 0)),
        ],
        out_specs=pl.BlockSpec((D, BE), lambda i: (0, i)),
        out_shape=jax.ShapeDtypeStruct((D, E), jnp.float32),
    )(e, pc, Wne, bn2d)


def _final_body(aggT_ref, b2d_ref, glob_ref, w2a_ref, b2_ref, w2g_ref,
                wgg_ref, wgm_ref, bg_ref, xn_ref, u_ref):
    aggT = aggT_ref[...]
    xb = lax.dot_general(aggT, w2a_ref[...], (((0,), (1,)), ((), ())),
                         preferred_element_type=jnp.float32)
    oh = (b2d_ref[...] == lax.broadcasted_iota(jnp.int32, (N, NB), 1))
    oh = oh.astype(jnp.float32)
    G2 = lax.dot_general(glob_ref[...], w2g_ref[...], (((1,), (1,)), ((), ())),
                         preferred_element_type=jnp.float32)
    xn = xb + lax.dot_general(oh, G2, (((1,), (0,)), ((), ())),
                              preferred_element_type=jnp.float32) + b2_ref[...]
    xn = jnp.maximum(xn, 0.0)
    xn_ref[...] = xn

    S = lax.dot_general(oh, xn, (((0,), (0,)), ((), ())),
                        preferred_element_type=jnp.float32)
    cnt = lax.dot_general(oh, jnp.ones((N, D), jnp.float32),
                          (((0,), (0,)), ((), ())),
                          preferred_element_type=jnp.float32)
    mean = S / jnp.maximum(cnt, 1.0)
    u = (lax.dot_general(glob_ref[...], wgg_ref[...],
                         (((1,), (1,)), ((), ())),
                         preferred_element_type=jnp.float32)
         + lax.dot_general(mean, wgm_ref[...], (((1,), (1,)), ((), ())),
                           preferred_element_type=jnp.float32)
         + bg_ref[...])
    u_ref[...] = jnp.maximum(u, 0.0)


def _k5(aggT, batch2d, glob, W2a, b22d, W2g, Wgg, Wgm, bg2d):
    return pl.pallas_call(
        _final_body,
        out_shape=[
            jax.ShapeDtypeStruct((N, D), jnp.float32),
            jax.ShapeDtypeStruct((NB, 32), jnp.float32),
        ],
    )(aggT, batch2d, glob, W2a, b22d, W2g, Wgg, Wgm, bg2d)


def _sc_mesh():
    return plsc.VectorSubcoreMesh(core_axis_name="c", subcore_axis_name="s")


_SC_PARAMS = pltpu.CompilerParams(needs_layout_passes=False,
                                  use_tc_tiling_on_sc=False)


def _edge_sc_body(q_hbm, p_hbm, row_hbm, col_hbm, r_hbm, e_hbm, pc_hbm,
                  idxr0, idxc0, qrows0, prows0, rbuf0,
                  idxr1, idxc1, qrows1, prows1, rbuf1,
                  semi0, semg0, semo0, semi1, semg1, semo1):
    wid = lax.axis_index("s") * NC + lax.axis_index("c")
    base0 = wid * (E // NW)
    nchunks = (E // NW) // C2
    slots = ((idxr0, idxc0, qrows0, prows0, rbuf0, semi0, semg0, semo0),
             (idxr1, idxc1, qrows1, prows1, rbuf1, semi1, semg1, semo1))

    def issue_idx(ci, s):
        idxr, idxc, _, _, _, semi, _, _ = slots[s]
        base = base0 + ci * C2
        pltpu.async_copy(row_hbm.at[pl.ds(base, C2)], idxr, semi)
        pltpu.async_copy(col_hbm.at[pl.ds(base, C2)], idxc, semi)

    def wait_idx(ci, s):
        idxr, idxc, _, _, _, semi, _, _ = slots[s]
        base = base0 + ci * C2
        pltpu.make_async_copy(row_hbm.at[pl.ds(base, C2)], idxr, semi).wait()
        pltpu.make_async_copy(col_hbm.at[pl.ds(base, C2)], idxc, semi).wait()

    def issue_g(ci, s):
        idxr, idxc, qrows, prows, rbuf, _, semg, _ = slots[s]
        base = base0 + ci * C2
        pltpu.async_copy(q_hbm.at[idxr], qrows, semg)
        pltpu.async_copy(p_hbm.at[idxc], prows, semg)
        pltpu.async_copy(r_hbm.at[pl.ds(base, C2)], rbuf, semg)

    def wait_g(ci, s):
        idxr, idxc, qrows, prows, rbuf, _, semg, _ = slots[s]
        base = base0 + ci * C2
        pltpu.make_async_copy(q_hbm.at[idxr], qrows, semg).wait()
        pltpu.make_async_copy(p_hbm.at[idxc], prows, semg).wait()
        pltpu.make_async_copy(r_hbm.at[pl.ds(base, C2)], rbuf, semg).wait()

    def issue_out(ci, s):
        _, _, qrows, prows, _, _, _, semo = slots[s]
        base = base0 + ci * C2
        pltpu.async_copy(qrows, e_hbm.at[pl.ds(base, C2)], semo)
        pltpu.async_copy(prows, pc_hbm.at[pl.ds(base, C2)], semo)

    def wait_out(ci, s):
        _, _, qrows, prows, _, _, _, semo = slots[s]
        base = base0 + ci * C2
        pltpu.make_async_copy(qrows, e_hbm.at[pl.ds(base, C2)], semo).wait()
        pltpu.make_async_copy(prows, pc_hbm.at[pl.ds(base, C2)], semo).wait()

    issue_idx(0, 0)
    issue_idx(1, 1)
    wait_idx(0, 0)
    issue_g(0, 0)

    def pair(i, carry):
        for s in (0, 1):
            ci = 2 * i + s
            so = 1 - s

            @pl.when((ci + 1 < nchunks) & (ci >= 1))
            def _():
                wait_out(ci - 1, so)

            @pl.when(ci + 1 < nchunks)
            def _():
                wait_idx(ci + 1, so)
                issue_g(ci + 1, so)

            wait_g(ci, s)
            _, _, qrows, prows, rbuf, _, _, _ = slots[s]

            def rowloop(ii, c2):
                for j in range(D // L):
                    slc = pl.ds(j * L, L)
                    v = qrows[ii, slc] + rbuf[ii, slc]
                    qrows[ii, slc] = jnp.maximum(v, 0.0)
                return c2

            lax.fori_loop(0, C2, rowloop, 0)
            issue_out(ci, s)

            @pl.when(ci + 2 < nchunks)
            def _():
                issue_idx(ci + 2, s)
        return carry

    lax.fori_loop(0, nchunks // 2, pair, 0)
    wait_out(nchunks - 2, 0)
    wait_out(nchunks - 1, 1)


def _k2(q, p, row, col, r):
    fn = pl.kernel(
        _edge_sc_body,
        out_type=[
            jax.ShapeDtypeStruct((E, D), jnp.float32),
            jax.ShapeDtypeStruct((E, D), jnp.float32),
        ],
        mesh=_sc_mesh(),
        compiler_params=_SC_PARAMS,
        scratch_types=[
            pltpu.VMEM((C2,), jnp.int32),
            pltpu.VMEM((C2,), jnp.int32),
            pltpu.VMEM((C2, D), jnp.float32),
            pltpu.VMEM((C2, D), jnp.float32),
            pltpu.VMEM((C2, D), jnp.float32),
            pltpu.VMEM((C2,), jnp.int32),
            pltpu.VMEM((C2,), jnp.int32),
            pltpu.VMEM((C2, D), jnp.float32),
            pltpu.VMEM((C2, D), jnp.float32),
            pltpu.VMEM((C2, D), jnp.float32),
            pltpu.SemaphoreType.DMA,
            pltpu.SemaphoreType.DMA,
            pltpu.SemaphoreType.DMA,
            pltpu.SemaphoreType.DMA,
            pltpu.SemaphoreType.DMA,
            pltpu.SemaphoreType.DMA,
        ],
    )
    return fn(q, p, row, col, r)


def _segmax_body(hT_hbm, row_hbm, aggT_hbm, agg_v, tmp_v,
                 hbuf0, rowb0, hbuf1, rowb1, mbuf, shr, sem0, sem1):
    c = lax.axis_index("c")
    s = lax.axis_index("s")
    pid = s // 2
    eh = s % 2
    f0 = (c * 8 + pid) * FPT
    ebase = eh * (E // EH)
    nchunks = (E // EH) // C4
    slots = ((hbuf0, rowb0, sem0), (hbuf1, rowb1, sem1))

    def issue(ci, slot):
        hbuf, rowb, sem = slots[slot]
        base = ebase + ci * C4
        pltpu.async_copy(row_hbm.at[pl.ds(base, C4)], rowb, sem)
        for f in range(FPT):
            pltpu.async_copy(hT_hbm.at[f0 + f, pl.ds(base, C4)],
                             hbuf.at[pl.ds(f * C4, C4)], sem)

    def wait(ci, slot):
        hbuf, rowb, sem = slots[slot]
        base = ebase + ci * C4
        pltpu.make_async_copy(row_hbm.at[pl.ds(base, C4)], rowb, sem).wait()
        for f in range(FPT):
            pltpu.make_async_copy(hT_hbm.at[f0 + f, pl.ds(base, C4)],
                                  hbuf.at[pl.ds(f * C4, C4)], sem).wait()

    issue(0, 0)
    issue(1, 1)

    neg = jnp.full((L,), _NEG_INF, jnp.float32)

    def initloop(i, carry):
        agg_v[pl.ds(i * L, L)] = neg
        return carry

    lax.fori_loop(0, (FPT * N) // L, initloop, 0)

    ids = lax.broadcasted_iota(jnp.int32, (L,), 0)

    def process(hbuf, rowb):
        def load_group(k):
            sl = pl.ds(k * L, L)
            rv = rowb[sl]
            plsc.store_scatter(tmp_v, [rv], ids)
            got = plsc.load_gather(tmp_v, [rv])
            bad = got != ids
            hvs = []
            for f in range(FPT):
                hvs.append(hbuf[pl.ds(f * C4 + k * L, L)])
            return rv, bad, hvs

        def fast_group(rv, hvs):
            ris = [rv + (f * N) for f in range(FPT)]
            curs = [plsc.load_gather(agg_v, [ris[f]]) for f in range(FPT)]
            for f in range(FPT):
                plsc.store_scatter(agg_v, [ris[f]],
                                   jnp.maximum(curs[f], hvs[f]))

        def slow_group(rv, hvs):
            for f in range(FPT):
                ri = rv + (f * N)
                hv = hvs[f]

                def cond(m):
                    return jnp.any(m)

                def body(m):
                    cur = plsc.load_gather(agg_v, [ri])
                    val = jnp.maximum(cur, hv)
                    plsc.store_scatter(agg_v, [ri], val, mask=m)
                    chk = plsc.load_gather(agg_v, [ri])
                    return m & (chk < hv)

                lax.while_loop(cond, body, jnp.ones((L,), jnp.bool_))

        def group2(g, c2):
            rv0, bad0, hvs0 = load_group(2 * g)
            rv1, bad1, hvs1 = load_group(2 * g + 1)
            has_dup = jnp.any(bad0 | bad1)

            def fast(_):
                fast_group(rv0, hvs0)
                fast_group(rv1, hvs1)
                return 0

            def slow(_):
                slow_group(rv0, hvs0)
                slow_group(rv1, hvs1)
                return 0

            lax.cond(has_dup, slow, fast, 0)
            return c2

        lax.fori_loop(0, C4 // L // 2, group2, 0)

    def pair(i, carry):
        for slot in (0, 1):
            ci = 2 * i + slot
            wait(ci, slot)
            hbuf, rowb, _ = slots[slot]
            process(hbuf, rowb)

            @pl.when(ci + 2 < nchunks)
            def _():
                issue(ci + 2, slot)
        return carry

    lax.fori_loop(0, nchunks // 2, pair, 0)

    def mergeloop(j, carry):
        @pl.when(eh == 1)
        def _():
            pltpu.sync_copy(agg_v.at[pl.ds(j * MP, MP)], shr.at[pid])

        plsc.subcore_barrier()

        @pl.when(eh == 0)
        def _():
            pltpu.sync_copy(shr.at[pid], mbuf)

            def vloop(i, c2):
                slc = pl.ds(j * MP + i * L, L)
                agg_v[slc] = jnp.maximum(agg_v[slc], mbuf[pl.ds(i * L, L)])
                return c2

            lax.fori_loop(0, MP // L, vloop, 0)

        plsc.subcore_barrier()
        return carry

    lax.fori_loop(0, (FPT * N) // MP, mergeloop, 0)

    @pl.when(eh == 0)
    def _():
        def fixloop(i, carry):
            sl = pl.ds(i * L, L)
            v = agg_v[sl]
            ok = (v - v) == 0.0
            agg_v[sl] = jnp.where(ok, v, 0.0)
            return carry

        lax.fori_loop(0, (FPT * N) // L, fixloop, 0)
        for f in range(FPT):
            pltpu.sync_copy(agg_v.at[pl.ds(f * N, N)], aggT_hbm.at[f0 + f])


def _k4(hT, row):
    fn = pl.kernel(
        _segmax_body,
        out_type=jax.ShapeDtypeStruct((D, N), jnp.float32),
        mesh=_sc_mesh(),
        compiler_params=_SC_PARAMS,
        scratch_types=[
            pltpu.VMEM((FPT * N,), jnp.float32),
            pltpu.VMEM((N,), jnp.int32),
            pltpu.VMEM((FPT * C4,), jnp.float32),
            pltpu.VMEM((C4,), jnp.int32),
            pltpu.VMEM((FPT * C4,), jnp.float32),
            pltpu.VMEM((C4,), jnp.int32),
            pltpu.VMEM((MP,), jnp.float32),
            pltpu.VMEM_SHARED((8, MP), jnp.float32),
            pltpu.SemaphoreType.DMA,
            pltpu.SemaphoreType.DMA,
        ],
    )
    return fn(hT, row)


def kernel(x, edge_index, edge_attr, glob, batch,
           W_edge, b_edge, W_node, b_node, W_node2, b_node2, W_glob, b_glob):
    row = edge_index[0]
    col = edge_index[1]
    Wex, Wea = W_edge[:, :D], W_edge[:, D:]
    Wnx, Wne = W_node[:, :D], W_node[:, D:]
    W2a, W2g = W_node2[:, :D], W_node2[:, D:]
    Wgg, Wgm = W_glob[:, :32], W_glob[:, 32:]
    be2d = b_edge.reshape(1, D)
    bn2d = b_node.reshape(D, 1)
    b22d = b_node2.reshape(1, D)
    bg2d = b_glob.reshape(1, 32)
    batch2d = batch.reshape(N, 1)

    q, p = _k1a(x, Wex, Wnx)
    r = _k1b(edge_attr, Wea, be2d)
    e, pc = _k2(q, p, row, col, r)
    hT = _k3(e, pc, Wne, bn2d)
    aggT = _k4(hT, row)
    xn, u = _k5(aggT, batch2d, glob, W2a, b22d, W2g, Wgg, Wgm, bg2d)
    return (xn, e, u)


# speedup vs baseline: 4.3983x; 1.0441x over previous
import functools
import jax
import jax.numpy as jnp
from jax import lax
from jax.experimental import pallas as pl
from jax.experimental.pallas import tpu as pltpu
from jax.experimental.pallas import tpu_sc as plsc

N = 10000
E = 320000
D = 128
NB = 16
NC = 2
NS = 16
NW = NC * NS
L = 16

BE = 2560
C2 = 40
C4 = 1280
FPT = 8
EH = 2
MP = 10000

_NEG_INF = float("-inf")


def _proj_body(x_ref, wex_ref, wnx_ref, q_ref, p_ref):
    xb = x_ref[...]
    q_ref[...] = lax.dot_general(xb, wex_ref[...], (((1,), (1,)), ((), ())),
                                 preferred_element_type=jnp.float32)
    p_ref[...] = lax.dot_general(xb, wnx_ref[...], (((1,), (1,)), ((), ())),
                                 preferred_element_type=jnp.float32)


def _k1a(x, Wex, Wnx):
    return pl.pallas_call(
        _proj_body,
        out_shape=[
            jax.ShapeDtypeStruct((N, D), jnp.float32),
            jax.ShapeDtypeStruct((N, D), jnp.float32),
        ],
    )(x, Wex, Wnx)


def _r_body(ea_ref, wea_ref, be_ref, rowg_ref, r_ref, flag_ref):
    r_ref[...] = lax.dot_general(ea_ref[...], wea_ref[...],
                                 (((1,), (1,)), ((), ())),
                                 preferred_element_type=jnp.float32) + be_ref[...]
    rg = rowg_ref[...]
    dup = None
    for sh in range(1, 16):
        c = jnp.concatenate([rg[:, sh:], rg[:, :sh]], axis=1)
        m = rg == c
        dup = m if dup is None else (dup | m)
    dupg = jnp.any(dup, axis=1)
    pairs = dupg.reshape(BE // 32, 2)
    flag_ref[...] = (pairs[:, 0] | pairs[:, 1]).astype(jnp.int32) \
        .reshape(BE // 32, 1)


def _k1b(edge_attr, Wea, be2d, rowg):
    return pl.pallas_call(
        _r_body,
        grid=(E // BE,),
        in_specs=[
            pl.BlockSpec((BE, 16), lambda i: (i, 0)),
            pl.BlockSpec((D, 16), lambda i: (0, 0)),
            pl.BlockSpec((1, D), lambda i: (0, 0)),
            pl.BlockSpec((BE // 16, 16), lambda i: (i, 0)),
        ],
        out_specs=[
            pl.BlockSpec((BE, D), lambda i: (i, 0)),
            pl.BlockSpec((BE // 32, 1), lambda i: (i, 0)),
        ],
        out_shape=[
            jax.ShapeDtypeStruct((E, D), jnp.float32),
            jax.ShapeDtypeStruct((E // 32, 1), jnp.int32),
        ],
    )(edge_attr, Wea, be2d, rowg)


def _hT_body(e_ref, pc_ref, wne_ref, bn_ref, hT_ref):
    hT_ref[...] = (lax.dot_general(wne_ref[...], e_ref[...],
                                   (((1,), (1,)), ((), ())),
                                   preferred_element_type=jnp.float32)
                   + lax.transpose(pc_ref[...], (1, 0)) + bn_ref[...])


def _k3(e, pc, Wne, bn2d):
    return pl.pallas_call(
        _hT_body,
        grid=(E // BE,),
        in_specs=[
            pl.BlockSpec((BE, D), lambda i: (i, 0)),
            pl.BlockSpec((BE, D), lambda i: (i, 0)),
            pl.BlockSpec((D, D), lambda i: (0, 0)),
            pl.BlockSpec((D, 1), lambda i: (0, 0)),
        ],
        out_specs=pl.BlockSpec((D, BE), lambda i: (0, i)),
        out_shape=jax.ShapeDtypeStruct((D, E), jnp.float32),
    )(e, pc, Wne, bn2d)


def _final_body(aggT_ref, b2d_ref, glob_ref, w2a_ref, b2_ref, w2g_ref,
                wgg_ref, wgm_ref, bg_ref, xn_ref, u_ref):
    aggT = aggT_ref[...]
    xb = lax.dot_general(aggT, w2a_ref[...], (((0,), (1,)), ((), ())),
                         preferred_element_type=jnp.float32)
    oh = (b2d_ref[...] == lax.broadcasted_iota(jnp.int32, (N, NB), 1))
    oh = oh.astype(jnp.float32)
    G2 = lax.dot_general(glob_ref[...], w2g_ref[...], (((1,), (1,)), ((), ())),
                         preferred_element_type=jnp.float32)
    xn = xb + lax.dot_general(oh, G2, (((1,), (0,)), ((), ())),
                              preferred_element_type=jnp.float32) + b2_ref[...]
    xn = jnp.maximum(xn, 0.0)
    xn_ref[...] = xn

    S = lax.dot_general(oh, xn, (((0,), (0,)), ((), ())),
                        preferred_element_type=jnp.float32)
    cnt = lax.dot_general(oh, jnp.ones((N, D), jnp.float32),
                          (((0,), (0,)), ((), ())),
                          preferred_element_type=jnp.float32)
    mean = S / jnp.maximum(cnt, 1.0)
    u = (lax.dot_general(glob_ref[...], wgg_ref[...],
                         (((1,), (1,)), ((), ())),
                         preferred_element_type=jnp.float32)
         + lax.dot_general(mean, wgm_ref[...], (((1,), (1,)), ((), ())),
                           preferred_element_type=jnp.float32)
         + bg_ref[...])
    u_ref[...] = jnp.maximum(u, 0.0)


def _k5(aggT, batch2d, glob, W2a, b22d, W2g, Wgg, Wgm, bg2d):
    return pl.pallas_call(
        _final_body,
        out_shape=[
            jax.ShapeDtypeStruct((N, D), jnp.float32),
            jax.ShapeDtypeStruct((NB, 32), jnp.float32),
        ],
    )(aggT, batch2d, glob, W2a, b22d, W2g, Wgg, Wgm, bg2d)


def _sc_mesh():
    return plsc.VectorSubcoreMesh(core_axis_name="c", subcore_axis_name="s")


_SC_PARAMS = pltpu.CompilerParams(needs_layout_passes=False,
                                  use_tc_tiling_on_sc=False)


def _edge_sc_body(q_hbm, p_hbm, row_hbm, col_hbm, r_hbm, e_hbm, pc_hbm,
                  idxr0, idxc0, qrows0, prows0, rbuf0,
                  idxr1, idxc1, qrows1, prows1, rbuf1,
                  semi0, semg0, semo0, semi1, semg1, semo1):
    wid = lax.axis_index("s") * NC + lax.axis_index("c")
    base0 = wid * (E // NW)
    nchunks = (E // NW) // C2
    slots = ((idxr0, idxc0, qrows0, prows0, rbuf0, semi0, semg0, semo0),
             (idxr1, idxc1, qrows1, prows1, rbuf1, semi1, semg1, semo1))

    def issue_idx(ci, s):
        idxr, idxc, _, _, _, semi, _, _ = slots[s]
        base = base0 + ci * C2
        pltpu.async_copy(row_hbm.at[pl.ds(base, C2)], idxr, semi)
        pltpu.async_copy(col_hbm.at[pl.ds(base, C2)], idxc, semi)

    def wait_idx(ci, s):
        idxr, idxc, _, _, _, semi, _, _ = slots[s]
        base = base0 + ci * C2
        pltpu.make_async_copy(row_hbm.at[pl.ds(base, C2)], idxr, semi).wait()
        pltpu.make_async_copy(col_hbm.at[pl.ds(base, C2)], idxc, semi).wait()

    def issue_g(ci, s):
        idxr, idxc, qrows, prows, rbuf, _, semg, _ = slots[s]
        base = base0 + ci * C2
        pltpu.async_copy(q_hbm.at[idxr], qrows, semg)
        pltpu.async_copy(p_hbm.at[idxc], prows, semg)
        pltpu.async_copy(r_hbm.at[pl.ds(base, C2)], rbuf, semg)

    def wait_g(ci, s):
        idxr, idxc, qrows, prows, rbuf, _, semg, _ = slots[s]
        base = base0 + ci * C2
        pltpu.make_async_copy(q_hbm.at[idxr], qrows, semg).wait()
        pltpu.make_async_copy(p_hbm.at[idxc], prows, semg).wait()
        pltpu.make_async_copy(r_hbm.at[pl.ds(base, C2)], rbuf, semg).wait()

    def issue_out(ci, s):
        _, _, qrows, prows, _, _, _, semo = slots[s]
        base = base0 + ci * C2
        pltpu.async_copy(qrows, e_hbm.at[pl.ds(base, C2)], semo)
        pltpu.async_copy(prows, pc_hbm.at[pl.ds(base, C2)], semo)

    def wait_out(ci, s):
        _, _, qrows, prows, _, _, _, semo = slots[s]
        base = base0 + ci * C2
        pltpu.make_async_copy(qrows, e_hbm.at[pl.ds(base, C2)], semo).wait()
        pltpu.make_async_copy(prows, pc_hbm.at[pl.ds(base, C2)], semo).wait()

    issue_idx(0, 0)
    issue_idx(1, 1)
    wait_idx(0, 0)
    issue_g(0, 0)

    def pair(i, carry):
        for s in (0, 1):
            ci = 2 * i + s
            so = 1 - s

            @pl.when((ci + 1 < nchunks) & (ci >= 1))
            def _():
                wait_out(ci - 1, so)

            @pl.when(ci + 1 < nchunks)
            def _():
                wait_idx(ci + 1, so)
                issue_g(ci + 1, so)

            wait_g(ci, s)
            _, _, qrows, prows, rbuf, _, _, _ = slots[s]

            def rowloop(ii, c2):
                for j in range(D // L):
                    slc = pl.ds(j * L, L)
                    v = qrows[ii, slc] + rbuf[ii, slc]
                    qrows[ii, slc] = jnp.maximum(v, 0.0)
                return c2

            lax.fori_loop(0, C2, rowloop, 0)
            issue_out(ci, s)

            @pl.when(ci + 2 < nchunks)
            def _():
                issue_idx(ci + 2, s)
        return carry

    lax.fori_loop(0, nchunks // 2, pair, 0)
    wait_out(nchunks - 2, 0)
    wait_out(nchunks - 1, 1)


def _k2(q, p, row, col, r):
    fn = pl.kernel(
        _edge_sc_body,
        out_type=[
            jax.ShapeDtypeStruct((E, D), jnp.float32),
            jax.ShapeDtypeStruct((E, D), jnp.float32),
        ],
        mesh=_sc_mesh(),
        compiler_params=_SC_PARAMS,
        scratch_types=[
            pltpu.VMEM((C2,), jnp.int32),
            pltpu.VMEM((C2,), jnp.int32),
            pltpu.VMEM((C2, D), jnp.float32),
            pltpu.VMEM((C2, D), jnp.float32),
            pltpu.VMEM((C2, D), jnp.float32),
            pltpu.VMEM((C2,), jnp.int32),
            pltpu.VMEM((C2,), jnp.int32),
            pltpu.VMEM((C2, D), jnp.float32),
            pltpu.VMEM((C2, D), jnp.float32),
            pltpu.VMEM((C2, D), jnp.float32),
            pltpu.SemaphoreType.DMA,
            pltpu.SemaphoreType.DMA,
            pltpu.SemaphoreType.DMA,
            pltpu.SemaphoreType.DMA,
            pltpu.SemaphoreType.DMA,
            pltpu.SemaphoreType.DMA,
        ],
    )
    return fn(q, p, row, col, r)


def _segmax_body(hT_hbm, row_hbm, dupf_hbm, aggT_hbm, agg_v,
                 hbuf0, rowb0, fbuf0, hbuf1, rowb1, fbuf1, mbuf, shr,
                 sem0, sem1):
    c = lax.axis_index("c")
    s = lax.axis_index("s")
    pid = s // 2
    eh = s % 2
    f0 = (c * 8 + pid) * FPT
    ebase = eh * (E // EH)
    fbase = eh * (E // EH // 32)
    NP = C4 // 32
    nchunks = (E // EH) // C4
    slots = ((hbuf0, rowb0, fbuf0, sem0), (hbuf1, rowb1, fbuf1, sem1))

    def issue(ci, slot):
        hbuf, rowb, fbuf, sem = slots[slot]
        base = ebase + ci * C4
        pltpu.async_copy(row_hbm.at[pl.ds(base, C4)], rowb, sem)
        pltpu.async_copy(dupf_hbm.at[pl.ds(fbase + ci * NP, NP)],
                         fbuf.at[pl.ds(0, NP)], sem)
        for f in range(FPT):
            pltpu.async_copy(hT_hbm.at[f0 + f, pl.ds(base, C4)],
                             hbuf.at[pl.ds(f * C4, C4)], sem)

    def wait(ci, slot):
        hbuf, rowb, fbuf, sem = slots[slot]
        base = ebase + ci * C4
        pltpu.make_async_copy(row_hbm.at[pl.ds(base, C4)], rowb, sem).wait()
        pltpu.make_async_copy(dupf_hbm.at[pl.ds(fbase + ci * NP, NP)],
                              fbuf.at[pl.ds(0, NP)], sem).wait()
        for f in range(FPT):
            pltpu.make_async_copy(hT_hbm.at[f0 + f, pl.ds(base, C4)],
                                  hbuf.at[pl.ds(f * C4, C4)], sem).wait()

    issue(0, 0)
    issue(1, 1)

    neg = jnp.full((L,), _NEG_INF, jnp.float32)

    def initloop(i, carry):
        agg_v[pl.ds(i * L, L)] = neg
        return carry

    lax.fori_loop(0, (FPT * N) // L, initloop, 0)

    def process(hbuf, rowb, fbuf):
        def load_group(k):
            sl = pl.ds(k * L, L)
            rv = rowb[sl]
            hvs = []
            for f in range(FPT):
                hvs.append(hbuf[pl.ds(f * C4 + k * L, L)])
            return rv, hvs

        def fast_group(rv, hvs):
            ris = [rv + (f * N) for f in range(FPT)]
            curs = [plsc.load_gather(agg_v, [ris[f]]) for f in range(FPT)]
            for f in range(FPT):
                plsc.store_scatter(agg_v, [ris[f]],
                                   jnp.maximum(curs[f], hvs[f]))

        def slow_group(rv, hvs):
            for f in range(FPT):
                ri = rv + (f * N)
                hv = hvs[f]

                def cond(m):
                    return jnp.any(m)

                def body(m):
                    cur = plsc.load_gather(agg_v, [ri])
                    val = jnp.maximum(cur, hv)
                    plsc.store_scatter(agg_v, [ri], val, mask=m)
                    chk = plsc.load_gather(agg_v, [ri])
                    return m & (chk < hv)

                lax.while_loop(cond, body, jnp.ones((L,), jnp.bool_))

        def group2(g, c2):
            rv0, hvs0 = load_group(2 * g)
            rv1, hvs1 = load_group(2 * g + 1)
            fv = fbuf[pl.ds(g, L)]
            has_dup = fv[0] != 0

            def fast(_):
                fast_group(rv0, hvs0)
                fast_group(rv1, hvs1)
                return 0

            def slow(_):
                slow_group(rv0, hvs0)
                slow_group(rv1, hvs1)
                return 0

            lax.cond(has_dup, slow, fast, 0)
            return c2

        lax.fori_loop(0, C4 // L // 2, group2, 0)

    def pair(i, carry):
        for slot in (0, 1):
            ci = 2 * i + slot
            wait(ci, slot)
            hbuf, rowb, fbuf, _ = slots[slot]
            process(hbuf, rowb, fbuf)

            @pl.when(ci + 2 < nchunks)
            def _():
                issue(ci + 2, slot)
        return carry

    lax.fori_loop(0, nchunks // 2, pair, 0)
    wait(nchunks - 1, 0)
    process(hbuf0, rowb0, fbuf0)

    def mergeloop(j, carry):
        @pl.when(eh == 1)
        def _():
            pltpu.sync_copy(agg_v.at[pl.ds(j * MP, MP)], shr.at[pid])

        plsc.subcore_barrier()

        @pl.when(eh == 0)
        def _():
            pltpu.sync_copy(shr.at[pid], mbuf)

            def vloop(i, c2):
                slc = pl.ds(j * MP + i * L, L)
                agg_v[slc] = jnp.maximum(agg_v[slc], mbuf[pl.ds(i * L, L)])
                return c2

            lax.fori_loop(0, MP // L, vloop, 0)

        plsc.subcore_barrier()
        return carry

    lax.fori_loop(0, (FPT * N) // MP, mergeloop, 0)

    @pl.when(eh == 0)
    def _():
        def fixloop(i, carry):
            sl = pl.ds(i * L, L)
            v = agg_v[sl]
            ok = (v - v) == 0.0
            agg_v[sl] = jnp.where(ok, v, 0.0)
            return carry

        lax.fori_loop(0, (FPT * N) // L, fixloop, 0)
        for f in range(FPT):
            pltpu.sync_copy(agg_v.at[pl.ds(f * N, N)], aggT_hbm.at[f0 + f])


def _k4(hT, row, dupf):
    fn = pl.kernel(
        _segmax_body,
        out_type=jax.ShapeDtypeStruct((D, N), jnp.float32),
        mesh=_sc_mesh(),
        compiler_params=_SC_PARAMS,
        scratch_types=[
            pltpu.VMEM((FPT * N,), jnp.float32),
            pltpu.VMEM((FPT * C4,), jnp.float32),
            pltpu.VMEM((C4,), jnp.int32),
            pltpu.VMEM((C4 // 32 + L,), jnp.int32),
            pltpu.VMEM((FPT * C4,), jnp.float32),
            pltpu.VMEM((C4,), jnp.int32),
            pltpu.VMEM((C4 // 32 + L,), jnp.int32),
            pltpu.VMEM((MP,), jnp.float32),
            pltpu.VMEM_SHARED((8, MP), jnp.float32),
            pltpu.SemaphoreType.DMA,
            pltpu.SemaphoreType.DMA,
        ],
    )
    return fn(hT, row, dupf)


def kernel(x, edge_index, edge_attr, glob, batch,
           W_edge, b_edge, W_node, b_node, W_node2, b_node2, W_glob, b_glob):
    row = edge_index[0]
    col = edge_index[1]
    Wex, Wea = W_edge[:, :D], W_edge[:, D:]
    Wnx, Wne = W_node[:, :D], W_node[:, D:]
    W2a, W2g = W_node2[:, :D], W_node2[:, D:]
    Wgg, Wgm = W_glob[:, :32], W_glob[:, 32:]
    be2d = b_edge.reshape(1, D)
    bn2d = b_node.reshape(D, 1)
    b22d = b_node2.reshape(1, D)
    bg2d = b_glob.reshape(1, 32)
    batch2d = batch.reshape(N, 1)

    q, p = _k1a(x, Wex, Wnx)
    rowg = row.reshape(E // 16, 16)
    r, dupf2d = _k1b(edge_attr, Wea, be2d, rowg)
    dupf = dupf2d.reshape(E // 32)
    e, pc = _k2(q, p, row, col, r)
    hT = _k3(e, pc, Wne, bn2d)
    aggT = _k4(hT, row, dupf)
    xn, u = _k5(aggT, batch2d, glob, W2a, b22d, W2g, Wgg, Wgm, bg2d)
    return (xn, e, u)


# speedup vs baseline: 4.6283x; 1.0523x over previous
import functools
import jax
import jax.numpy as jnp
from jax import lax
from jax.experimental import pallas as pl
from jax.experimental.pallas import tpu as pltpu
from jax.experimental.pallas import tpu_sc as plsc

N = 10000
E = 320000
D = 128
NB = 16
NC = 2
NS = 16
NW = NC * NS
L = 16

BE = 6400
C2 = 40
C4 = 1280
FPT = 8
EH = 2
MP = 10000

_NEG_INF = float("-inf")


def _proj_body(x_ref, wex_ref, wnx_ref, q_ref, p_ref):
    xb = x_ref[...]
    q_ref[...] = lax.dot_general(xb, wex_ref[...], (((1,), (1,)), ((), ())),
                                 preferred_element_type=jnp.float32)
    p_ref[...] = lax.dot_general(xb, wnx_ref[...], (((1,), (1,)), ((), ())),
                                 preferred_element_type=jnp.float32)


def _k1a(x, Wex, Wnx):
    return pl.pallas_call(
        _proj_body,
        out_shape=[
            jax.ShapeDtypeStruct((N, D), jnp.float32),
            jax.ShapeDtypeStruct((N, D), jnp.float32),
        ],
    )(x, Wex, Wnx)


def _r_body(ea_ref, wea_ref, be_ref, rowg_ref, r_ref, flag_ref):
    r_ref[...] = lax.dot_general(ea_ref[...], wea_ref[...],
                                 (((1,), (1,)), ((), ())),
                                 preferred_element_type=jnp.float32) + be_ref[...]
    rg = rowg_ref[...]
    dup = None
    for sh in range(1, 16):
        c = jnp.concatenate([rg[:, sh:], rg[:, :sh]], axis=1)
        m = rg == c
        dup = m if dup is None else (dup | m)
    dupg = jnp.any(dup, axis=1)
    pairs = dupg.reshape(BE // 32, 2)
    flag_ref[...] = (pairs[:, 0] | pairs[:, 1]).astype(jnp.int32) \
        .reshape(BE // 32, 1)


def _k1b(edge_attr, Wea, be2d, rowg):
    return pl.pallas_call(
        _r_body,
        grid=(E // BE,),
        in_specs=[
            pl.BlockSpec((BE, 16), lambda i: (i, 0)),
            pl.BlockSpec((D, 16), lambda i: (0, 0)),
            pl.BlockSpec((1, D), lambda i: (0, 0)),
            pl.BlockSpec((BE // 16, 16), lambda i: (i, 0)),
        ],
        out_specs=[
            pl.BlockSpec((BE, D), lambda i: (i, 0)),
            pl.BlockSpec((BE // 32, 1), lambda i: (i, 0)),
        ],
        out_shape=[
            jax.ShapeDtypeStruct((E, D), jnp.float32),
            jax.ShapeDtypeStruct((E // 32, 1), jnp.int32),
        ],
    )(edge_attr, Wea, be2d, rowg)


def _hT_body(e_ref, pc_ref, wne_ref, bn_ref, hT_ref):
    hT_ref[...] = (lax.dot_general(wne_ref[...], e_ref[...],
                                   (((1,), (1,)), ((), ())),
                                   preferred_element_type=jnp.float32)
                   + lax.transpose(pc_ref[...], (1, 0)) + bn_ref[...])


def _k3(e, pc, Wne, bn2d):
    return pl.pallas_call(
        _hT_body,
        grid=(E // BE,),
        in_specs=[
            pl.BlockSpec((BE, D), lambda i: (i, 0)),
            pl.BlockSpec((BE, D), lambda i: (i, 0)),
            pl.BlockSpec((D, D), lambda i: (0, 0)),
            pl.BlockSpec((D, 1), lambda i: (0, 0)),
        ],
        out_specs=pl.BlockSpec((D, BE), lambda i: (0, i)),
        out_shape=jax.ShapeDtypeStruct((D, E), jnp.float32),
    )(e, pc, Wne, bn2d)


def _final_body(aggT_ref, b2d_ref, glob_ref, w2a_ref, b2_ref, w2g_ref,
                wgg_ref, wgm_ref, bg_ref, xn_ref, u_ref):
    aggT = aggT_ref[...]
    xb = lax.dot_general(aggT, w2a_ref[...], (((0,), (1,)), ((), ())),
                         preferred_element_type=jnp.float32)
    oh = (b2d_ref[...] == lax.broadcasted_iota(jnp.int32, (N, NB), 1))
    oh = oh.astype(jnp.float32)
    G2 = lax.dot_general(glob_ref[...], w2g_ref[...], (((1,), (1,)), ((), ())),
                         preferred_element_type=jnp.float32)
    xn = xb + lax.dot_general(oh, G2, (((1,), (0,)), ((), ())),
                              preferred_element_type=jnp.float32) + b2_ref[...]
    xn = jnp.maximum(xn, 0.0)
    xn_ref[...] = xn

    S = lax.dot_general(oh, xn, (((0,), (0,)), ((), ())),
                        preferred_element_type=jnp.float32)
    cnt = lax.dot_general(oh, jnp.ones((N, D), jnp.float32),
                          (((0,), (0,)), ((), ())),
                          preferred_element_type=jnp.float32)
    mean = S / jnp.maximum(cnt, 1.0)
    u = (lax.dot_general(glob_ref[...], wgg_ref[...],
                         (((1,), (1,)), ((), ())),
                         preferred_element_type=jnp.float32)
         + lax.dot_general(mean, wgm_ref[...], (((1,), (1,)), ((), ())),
                           preferred_element_type=jnp.float32)
         + bg_ref[...])
    u_ref[...] = jnp.maximum(u, 0.0)


def _k5(aggT, batch2d, glob, W2a, b22d, W2g, Wgg, Wgm, bg2d):
    return pl.pallas_call(
        _final_body,
        out_shape=[
            jax.ShapeDtypeStruct((N, D), jnp.float32),
            jax.ShapeDtypeStruct((NB, 32), jnp.float32),
        ],
    )(aggT, batch2d, glob, W2a, b22d, W2g, Wgg, Wgm, bg2d)


def _sc_mesh():
    return plsc.VectorSubcoreMesh(core_axis_name="c", subcore_axis_name="s")


_SC_PARAMS = pltpu.CompilerParams(needs_layout_passes=False,
                                  use_tc_tiling_on_sc=False)


def _edge_sc_body(q_hbm, p_hbm, row_hbm, col_hbm, r_hbm, e_hbm, pc_hbm,
                  idxr0, idxc0, qrows0, prows0, rbuf0,
                  idxr1, idxc1, qrows1, prows1, rbuf1,
                  semi0, semg0, semo0, semi1, semg1, semo1):
    wid = lax.axis_index("s") * NC + lax.axis_index("c")
    base0 = wid * (E // NW)
    nchunks = (E // NW) // C2
    slots = ((idxr0, idxc0, qrows0, prows0, rbuf0, semi0, semg0, semo0),
             (idxr1, idxc1, qrows1, prows1, rbuf1, semi1, semg1, semo1))

    def issue_idx(ci, s):
        idxr, idxc, _, _, _, semi, _, _ = slots[s]
        base = base0 + ci * C2
        pltpu.async_copy(row_hbm.at[pl.ds(base, C2)], idxr, semi)
        pltpu.async_copy(col_hbm.at[pl.ds(base, C2)], idxc, semi)

    def wait_idx(ci, s):
        idxr, idxc, _, _, _, semi, _, _ = slots[s]
        base = base0 + ci * C2
        pltpu.make_async_copy(row_hbm.at[pl.ds(base, C2)], idxr, semi).wait()
        pltpu.make_async_copy(col_hbm.at[pl.ds(base, C2)], idxc, semi).wait()

    def issue_g(ci, s):
        idxr, idxc, qrows, prows, rbuf, _, semg, _ = slots[s]
        base = base0 + ci * C2
        pltpu.async_copy(q_hbm.at[idxr], qrows, semg)
        pltpu.async_copy(p_hbm.at[idxc], prows, semg)
        pltpu.async_copy(r_hbm.at[pl.ds(base, C2)], rbuf, semg)

    def wait_g(ci, s):
        idxr, idxc, qrows, prows, rbuf, _, semg, _ = slots[s]
        base = base0 + ci * C2
        pltpu.make_async_copy(q_hbm.at[idxr], qrows, semg).wait()
        pltpu.make_async_copy(p_hbm.at[idxc], prows, semg).wait()
        pltpu.make_async_copy(r_hbm.at[pl.ds(base, C2)], rbuf, semg).wait()

    def issue_out(ci, s):
        _, _, qrows, prows, _, _, _, semo = slots[s]
        base = base0 + ci * C2
        pltpu.async_copy(qrows, e_hbm.at[pl.ds(base, C2)], semo)
        pltpu.async_copy(prows, pc_hbm.at[pl.ds(base, C2)], semo)

    def wait_out(ci, s):
        _, _, qrows, prows, _, _, _, semo = slots[s]
        base = base0 + ci * C2
        pltpu.make_async_copy(qrows, e_hbm.at[pl.ds(base, C2)], semo).wait()
        pltpu.make_async_copy(prows, pc_hbm.at[pl.ds(base, C2)], semo).wait()

    issue_idx(0, 0)
    issue_idx(1, 1)
    wait_idx(0, 0)
    issue_g(0, 0)

    def pair(i, carry):
        for s in (0, 1):
            ci = 2 * i + s
            so = 1 - s

            @pl.when((ci + 1 < nchunks) & (ci >= 1))
            def _():
                wait_out(ci - 1, so)

            @pl.when(ci + 1 < nchunks)
            def _():
                wait_idx(ci + 1, so)
                issue_g(ci + 1, so)

            wait_g(ci, s)
            _, _, qrows, prows, rbuf, _, _, _ = slots[s]

            def rowloop(ii, c2):
                for j in range(D // L):
                    slc = pl.ds(j * L, L)
                    v = qrows[ii, slc] + rbuf[ii, slc]
                    qrows[ii, slc] = jnp.maximum(v, 0.0)
                return c2

            lax.fori_loop(0, C2, rowloop, 0)
            issue_out(ci, s)

            @pl.when(ci + 2 < nchunks)
            def _():
                issue_idx(ci + 2, s)
        return carry

    lax.fori_loop(0, nchunks // 2, pair, 0)
    wait_out(nchunks - 2, 0)
    wait_out(nchunks - 1, 1)


def _k2(q, p, row, col, r):
    fn = pl.kernel(
        _edge_sc_body,
        out_type=[
            jax.ShapeDtypeStruct((E, D), jnp.float32),
            jax.ShapeDtypeStruct((E, D), jnp.float32),
        ],
        mesh=_sc_mesh(),
        compiler_params=_SC_PARAMS,
        scratch_types=[
            pltpu.VMEM((C2,), jnp.int32),
            pltpu.VMEM((C2,), jnp.int32),
            pltpu.VMEM((C2, D), jnp.float32),
            pltpu.VMEM((C2, D), jnp.float32),
            pltpu.VMEM((C2, D), jnp.float32),
            pltpu.VMEM((C2,), jnp.int32),
            pltpu.VMEM((C2,), jnp.int32),
            pltpu.VMEM((C2, D), jnp.float32),
            pltpu.VMEM((C2, D), jnp.float32),
            pltpu.VMEM((C2, D), jnp.float32),
            pltpu.SemaphoreType.DMA,
            pltpu.SemaphoreType.DMA,
            pltpu.SemaphoreType.DMA,
            pltpu.SemaphoreType.DMA,
            pltpu.SemaphoreType.DMA,
            pltpu.SemaphoreType.DMA,
        ],
    )
    return fn(q, p, row, col, r)


def _segmax_body(hT_hbm, row_hbm, dupf_hbm, aggT_hbm, agg_v,
                 hbuf0, rowb0, fbuf0, hbuf1, rowb1, fbuf1, mbuf, shr,
                 sem0, sem1):
    c = lax.axis_index("c")
    s = lax.axis_index("s")
    pid = s // 2
    eh = s % 2
    f0 = (c * 8 + pid) * FPT
    ebase = eh * (E // EH)
    fbase = eh * (E // EH // 32)
    NP = C4 // 32
    nchunks = (E // EH) // C4
    slots = ((hbuf0, rowb0, fbuf0, sem0), (hbuf1, rowb1, fbuf1, sem1))

    def issue(ci, slot):
        hbuf, rowb, fbuf, sem = slots[slot]
        base = ebase + ci * C4
        pltpu.async_copy(row_hbm.at[pl.ds(base, C4)], rowb, sem)
        pltpu.async_copy(dupf_hbm.at[pl.ds(fbase + ci * NP, NP)],
                         fbuf.at[pl.ds(0, NP)], sem)
        for f in range(FPT):
            pltpu.async_copy(hT_hbm.at[f0 + f, pl.ds(base, C4)],
                             hbuf.at[pl.ds(f * C4, C4)], sem)

    def wait(ci, slot):
        hbuf, rowb, fbuf, sem = slots[slot]
        base = ebase + ci * C4
        pltpu.make_async_copy(row_hbm.at[pl.ds(base, C4)], rowb, sem).wait()
        pltpu.make_async_copy(dupf_hbm.at[pl.ds(fbase + ci * NP, NP)],
                              fbuf.at[pl.ds(0, NP)], sem).wait()
        for f in range(FPT):
            pltpu.make_async_copy(hT_hbm.at[f0 + f, pl.ds(base, C4)],
                                  hbuf.at[pl.ds(f * C4, C4)], sem).wait()

    issue(0, 0)
    issue(1, 1)

    neg = jnp.full((L,), _NEG_INF, jnp.float32)

    def initloop(i, carry):
        agg_v[pl.ds(i * L, L)] = neg
        return carry

    lax.fori_loop(0, (FPT * N) // L, initloop, 0)

    def process(hbuf, rowb, fbuf):
        def load_group(k):
            sl = pl.ds(k * L, L)
            rv = rowb[sl]
            hvs = []
            for f in range(FPT):
                hvs.append(hbuf[pl.ds(f * C4 + k * L, L)])
            return rv, hvs

        def fast_group(rv, hvs):
            ris = [rv + (f * N) for f in range(FPT)]
            curs = [plsc.load_gather(agg_v, [ris[f]]) for f in range(FPT)]
            for f in range(FPT):
                plsc.store_scatter(agg_v, [ris[f]],
                                   jnp.maximum(curs[f], hvs[f]))

        def slow_group(rv, hvs):
            for f in range(FPT):
                ri = rv + (f * N)
                hv = hvs[f]

                def cond(m):
                    return jnp.any(m)

                def body(m):
                    cur = plsc.load_gather(agg_v, [ri])
                    val = jnp.maximum(cur, hv)
                    plsc.store_scatter(agg_v, [ri], val, mask=m)
                    chk = plsc.load_gather(agg_v, [ri])
                    return m & (chk < hv)

                lax.while_loop(cond, body, jnp.ones((L,), jnp.bool_))

        def group2(g, c2):
            rv0, hvs0 = load_group(2 * g)
            rv1, hvs1 = load_group(2 * g + 1)
            fv = fbuf[pl.ds(g, L)]
            has_dup = fv[0] != 0

            def fast(_):
                fast_group(rv0, hvs0)
                fast_group(rv1, hvs1)
                return 0

            def slow(_):
                slow_group(rv0, hvs0)
                slow_group(rv1, hvs1)
                return 0

            lax.cond(has_dup, slow, fast, 0)
            return c2

        lax.fori_loop(0, C4 // L // 2, group2, 0)

    def pair(i, carry):
        for slot in (0, 1):
            ci = 2 * i + slot
            wait(ci, slot)
            hbuf, rowb, fbuf, _ = slots[slot]
            process(hbuf, rowb, fbuf)

            @pl.when(ci + 2 < nchunks)
            def _():
                issue(ci + 2, slot)
        return carry

    lax.fori_loop(0, nchunks // 2, pair, 0)
    wait(nchunks - 1, 0)
    process(hbuf0, rowb0, fbuf0)

    def mergeloop(j, carry):
        @pl.when(eh == 1)
        def _():
            pltpu.sync_copy(agg_v.at[pl.ds(j * MP, MP)], shr.at[pid])

        plsc.subcore_barrier()

        @pl.when(eh == 0)
        def _():
            pltpu.sync_copy(shr.at[pid], mbuf)

            def vloop(i, c2):
                slc = pl.ds(j * MP + i * L, L)
                agg_v[slc] = jnp.maximum(agg_v[slc], mbuf[pl.ds(i * L, L)])
                return c2

            lax.fori_loop(0, MP // L, vloop, 0)

        plsc.subcore_barrier()
        return carry

    lax.fori_loop(0, (FPT * N) // MP, mergeloop, 0)

    @pl.when(eh == 0)
    def _():
        def fixloop(i, carry):
            sl = pl.ds(i * L, L)
            v = agg_v[sl]
            ok = (v - v) == 0.0
            agg_v[sl] = jnp.where(ok, v, 0.0)
            return carry

        lax.fori_loop(0, (FPT * N) // L, fixloop, 0)
        for f in range(FPT):
            pltpu.sync_copy(agg_v.at[pl.ds(f * N, N)], aggT_hbm.at[f0 + f])


def _k4(hT, row, dupf):
    fn = pl.kernel(
        _segmax_body,
        out_type=jax.ShapeDtypeStruct((D, N), jnp.float32),
        mesh=_sc_mesh(),
        compiler_params=_SC_PARAMS,
        scratch_types=[
            pltpu.VMEM((FPT * N,), jnp.float32),
            pltpu.VMEM((FPT * C4,), jnp.float32),
            pltpu.VMEM((C4,), jnp.int32),
            pltpu.VMEM((C4 // 32 + L,), jnp.int32),
            pltpu.VMEM((FPT * C4,), jnp.float32),
            pltpu.VMEM((C4,), jnp.int32),
            pltpu.VMEM((C4 // 32 + L,), jnp.int32),
            pltpu.VMEM((MP,), jnp.float32),
            pltpu.VMEM_SHARED((8, MP), jnp.float32),
            pltpu.SemaphoreType.DMA,
            pltpu.SemaphoreType.DMA,
        ],
    )
    return fn(hT, row, dupf)


def kernel(x, edge_index, edge_attr, glob, batch,
           W_edge, b_edge, W_node, b_node, W_node2, b_node2, W_glob, b_glob):
    row = edge_index[0]
    col = edge_index[1]
    Wex, Wea = W_edge[:, :D], W_edge[:, D:]
    Wnx, Wne = W_node[:, :D], W_node[:, D:]
    W2a, W2g = W_node2[:, :D], W_node2[:, D:]
    Wgg, Wgm = W_glob[:, :32], W_glob[:, 32:]
    be2d = b_edge.reshape(1, D)
    bn2d = b_node.reshape(D, 1)
    b22d = b_node2.reshape(1, D)
    bg2d = b_glob.reshape(1, 32)
    batch2d = batch.reshape(N, 1)

    q, p = _k1a(x, Wex, Wnx)
    rowg = row.reshape(E // 16, 16)
    r, dupf2d = _k1b(edge_attr, Wea, be2d, rowg)
    dupf = dupf2d.reshape(E // 32)
    e, pc = _k2(q, p, row, col, r)
    hT = _k3(e, pc, Wne, bn2d)
    aggT = _k4(hT, row, dupf)
    xn, u = _k5(aggT, batch2d, glob, W2a, b22d, W2g, Wgg, Wgm, bg2d)
    return (xn, e, u)


# speedup vs baseline: 4.6374x; 1.0020x over previous
import functools
import jax
import jax.numpy as jnp
from jax import lax
from jax.experimental import pallas as pl
from jax.experimental.pallas import tpu as pltpu
from jax.experimental.pallas import tpu_sc as plsc

N = 10000
E = 320000
D = 128
NB = 16
NC = 2
NS = 16
NW = NC * NS
L = 16

BE = 6400
C2 = 40
C4 = 1280
FPT = 8
EH = 2
MP = 10000

_NEG_INF = float("-inf")


def _r_body(ea_ref, wea_ref, be_ref, rowg_ref, x_ref, wex_ref, wnx_ref,
            r_ref, flag_ref, q_ref, p_ref):
    @pl.when(pl.program_id(0) == 0)
    def _():
        xb = x_ref[...]
        q_ref[...] = lax.dot_general(xb, wex_ref[...], (((1,), (1,)), ((), ())),
                                     preferred_element_type=jnp.float32)
        p_ref[...] = lax.dot_general(xb, wnx_ref[...], (((1,), (1,)), ((), ())),
                                     preferred_element_type=jnp.float32)

    r_ref[...] = lax.dot_general(ea_ref[...], wea_ref[...],
                                 (((1,), (1,)), ((), ())),
                                 preferred_element_type=jnp.float32) + be_ref[...]
    rg = rowg_ref[...]
    dup = None
    for sh in range(1, 16):
        c = jnp.concatenate([rg[:, sh:], rg[:, :sh]], axis=1)
        m = rg == c
        dup = m if dup is None else (dup | m)
    dupg = jnp.any(dup, axis=1)
    pairs = dupg.reshape(BE // 32, 2)
    flag_ref[...] = (pairs[:, 0] | pairs[:, 1]).astype(jnp.int32) \
        .reshape(BE // 32, 1)


def _k1b(edge_attr, Wea, be2d, rowg, x, Wex, Wnx):
    return pl.pallas_call(
        _r_body,
        grid=(E // BE,),
        in_specs=[
            pl.BlockSpec((BE, 16), lambda i: (i, 0)),
            pl.BlockSpec((D, 16), lambda i: (0, 0)),
            pl.BlockSpec((1, D), lambda i: (0, 0)),
            pl.BlockSpec((BE // 16, 16), lambda i: (i, 0)),
            pl.BlockSpec((N, D), lambda i: (0, 0)),
            pl.BlockSpec((D, D), lambda i: (0, 0)),
            pl.BlockSpec((D, D), lambda i: (0, 0)),
        ],
        out_specs=[
            pl.BlockSpec((BE, D), lambda i: (i, 0)),
            pl.BlockSpec((BE // 32, 1), lambda i: (i, 0)),
            pl.BlockSpec((N, D), lambda i: (0, 0)),
            pl.BlockSpec((N, D), lambda i: (0, 0)),
        ],
        out_shape=[
            jax.ShapeDtypeStruct((E, D), jnp.float32),
            jax.ShapeDtypeStruct((E // 32, 1), jnp.int32),
            jax.ShapeDtypeStruct((N, D), jnp.float32),
            jax.ShapeDtypeStruct((N, D), jnp.float32),
        ],
    )(edge_attr, Wea, be2d, rowg, x, Wex, Wnx)


def _hT_body(e_ref, pc_ref, wne_ref, bn_ref, hT_ref):
    hT_ref[...] = (lax.dot_general(wne_ref[...], e_ref[...],
                                   (((1,), (1,)), ((), ())),
                                   preferred_element_type=jnp.float32)
                   + lax.transpose(pc_ref[...], (1, 0)) + bn_ref[...])


def _k3(e, pc, Wne, bn2d):
    return pl.pallas_call(
        _hT_body,
        grid=(E // BE,),
        in_specs=[
            pl.BlockSpec((BE, D), lambda i: (i, 0)),
            pl.BlockSpec((BE, D), lambda i: (i, 0)),
            pl.BlockSpec((D, D), lambda i: (0, 0)),
            pl.BlockSpec((D, 1), lambda i: (0, 0)),
        ],
        out_specs=pl.BlockSpec((D, BE), lambda i: (0, i)),
        out_shape=jax.ShapeDtypeStruct((D, E), jnp.float32),
    )(e, pc, Wne, bn2d)


def _final_body(aggT_ref, b2d_ref, glob_ref, w2a_ref, b2_ref, w2g_ref,
                wgg_ref, wgm_ref, bg_ref, xn_ref, u_ref):
    aggT = aggT_ref[...]
    xb = lax.dot_general(aggT, w2a_ref[...], (((0,), (1,)), ((), ())),
                         preferred_element_type=jnp.float32)
    oh = (b2d_ref[...] == lax.broadcasted_iota(jnp.int32, (N, NB), 1))
    oh = oh.astype(jnp.float32)
    G2 = lax.dot_general(glob_ref[...], w2g_ref[...], (((1,), (1,)), ((), ())),
                         preferred_element_type=jnp.float32)
    xn = xb + lax.dot_general(oh, G2, (((1,), (0,)), ((), ())),
                              preferred_element_type=jnp.float32) + b2_ref[...]
    xn = jnp.maximum(xn, 0.0)
    xn_ref[...] = xn

    S = lax.dot_general(oh, xn, (((0,), (0,)), ((), ())),
                        preferred_element_type=jnp.float32)
    cnt = lax.dot_general(oh, jnp.ones((N, D), jnp.float32),
                          (((0,), (0,)), ((), ())),
                          preferred_element_type=jnp.float32)
    mean = S / jnp.maximum(cnt, 1.0)
    u = (lax.dot_general(glob_ref[...], wgg_ref[...],
                         (((1,), (1,)), ((), ())),
                         preferred_element_type=jnp.float32)
         + lax.dot_general(mean, wgm_ref[...], (((1,), (1,)), ((), ())),
                           preferred_element_type=jnp.float32)
         + bg_ref[...])
    u_ref[...] = jnp.maximum(u, 0.0)


def _k5(aggT, batch2d, glob, W2a, b22d, W2g, Wgg, Wgm, bg2d):
    return pl.pallas_call(
        _final_body,
        out_shape=[
            jax.ShapeDtypeStruct((N, D), jnp.float32),
            jax.ShapeDtypeStruct((NB, 32), jnp.float32),
        ],
    )(aggT, batch2d, glob, W2a, b22d, W2g, Wgg, Wgm, bg2d)


def _sc_mesh():
    return plsc.VectorSubcoreMesh(core_axis_name="c", subcore_axis_name="s")


_SC_PARAMS = pltpu.CompilerParams(needs_layout_passes=False,
                                  use_tc_tiling_on_sc=False)


def _edge_sc_body(q_hbm, p_hbm, row_hbm, col_hbm, r_hbm, e_hbm, pc_hbm,
                  idxr0, idxc0, qrows0, prows0, rbuf0,
                  idxr1, idxc1, qrows1, prows1, rbuf1,
                  semi0, semg0, semo0, semi1, semg1, semo1):
    wid = lax.axis_index("s") * NC + lax.axis_index("c")
    base0 = wid * (E // NW)
    nchunks = (E // NW) // C2
    slots = ((idxr0, idxc0, qrows0, prows0, rbuf0, semi0, semg0, semo0),
             (idxr1, idxc1, qrows1, prows1, rbuf1, semi1, semg1, semo1))

    def issue_idx(ci, s):
        idxr, idxc, _, _, _, semi, _, _ = slots[s]
        base = base0 + ci * C2
        pltpu.async_copy(row_hbm.at[pl.ds(base, C2)], idxr, semi)
        pltpu.async_copy(col_hbm.at[pl.ds(base, C2)], idxc, semi)

    def wait_idx(ci, s):
        idxr, idxc, _, _, _, semi, _, _ = slots[s]
        base = base0 + ci * C2
        pltpu.make_async_copy(row_hbm.at[pl.ds(base, C2)], idxr, semi).wait()
        pltpu.make_async_copy(col_hbm.at[pl.ds(base, C2)], idxc, semi).wait()

    def issue_g(ci, s):
        idxr, idxc, qrows, prows, rbuf, _, semg, _ = slots[s]
        base = base0 + ci * C2
        pltpu.async_copy(q_hbm.at[idxr], qrows, semg)
        pltpu.async_copy(p_hbm.at[idxc], prows, semg)
        pltpu.async_copy(r_hbm.at[pl.ds(base, C2)], rbuf, semg)

    def wait_g(ci, s):
        idxr, idxc, qrows, prows, rbuf, _, semg, _ = slots[s]
        base = base0 + ci * C2
        pltpu.make_async_copy(q_hbm.at[idxr], qrows, semg).wait()
        pltpu.make_async_copy(p_hbm.at[idxc], prows, semg).wait()
        pltpu.make_async_copy(r_hbm.at[pl.ds(base, C2)], rbuf, semg).wait()

    def issue_out(ci, s):
        _, _, qrows, prows, _, _, _, semo = slots[s]
        base = base0 + ci * C2
        pltpu.async_copy(qrows, e_hbm.at[pl.ds(base, C2)], semo)
        pltpu.async_copy(prows, pc_hbm.at[pl.ds(base, C2)], semo)

    def wait_out(ci, s):
        _, _, qrows, prows, _, _, _, semo = slots[s]
        base = base0 + ci * C2
        pltpu.make_async_copy(qrows, e_hbm.at[pl.ds(base, C2)], semo).wait()
        pltpu.make_async_copy(prows, pc_hbm.at[pl.ds(base, C2)], semo).wait()

    issue_idx(0, 0)
    issue_idx(1, 1)
    wait_idx(0, 0)
    issue_g(0, 0)

    def pair(i, carry):
        for s in (0, 1):
            ci = 2 * i + s
            so = 1 - s

            @pl.when((ci + 1 < nchunks) & (ci >= 1))
            def _():
                wait_out(ci - 1, so)

            @pl.when(ci + 1 < nchunks)
            def _():
                wait_idx(ci + 1, so)
                issue_g(ci + 1, so)

            wait_g(ci, s)
            _, _, qrows, prows, rbuf, _, _, _ = slots[s]

            def rowloop(ii, c2):
                for j in range(D // L):
                    slc = pl.ds(j * L, L)
                    v = qrows[ii, slc] + rbuf[ii, slc]
                    qrows[ii, slc] = jnp.maximum(v, 0.0)
                return c2

            lax.fori_loop(0, C2, rowloop, 0)
            issue_out(ci, s)

            @pl.when(ci + 2 < nchunks)
            def _():
                issue_idx(ci + 2, s)
        return carry

    lax.fori_loop(0, nchunks // 2, pair, 0)
    wait_out(nchunks - 2, 0)
    wait_out(nchunks - 1, 1)


def _k2(q, p, row, col, r):
    fn = pl.kernel(
        _edge_sc_body,
        out_type=[
            jax.ShapeDtypeStruct((E, D), jnp.float32),
            jax.ShapeDtypeStruct((E, D), jnp.float32),
        ],
        mesh=_sc_mesh(),
        compiler_params=_SC_PARAMS,
        scratch_types=[
            pltpu.VMEM((C2,), jnp.int32),
            pltpu.VMEM((C2,), jnp.int32),
            pltpu.VMEM((C2, D), jnp.float32),
            pltpu.VMEM((C2, D), jnp.float32),
            pltpu.VMEM((C2, D), jnp.float32),
            pltpu.VMEM((C2,), jnp.int32),
            pltpu.VMEM((C2,), jnp.int32),
            pltpu.VMEM((C2, D), jnp.float32),
            pltpu.VMEM((C2, D), jnp.float32),
            pltpu.VMEM((C2, D), jnp.float32),
            pltpu.SemaphoreType.DMA,
            pltpu.SemaphoreType.DMA,
            pltpu.SemaphoreType.DMA,
            pltpu.SemaphoreType.DMA,
            pltpu.SemaphoreType.DMA,
            pltpu.SemaphoreType.DMA,
        ],
    )
    return fn(q, p, row, col, r)


def _segmax_body(hT_hbm, row_hbm, dupf_hbm, aggT_hbm, agg_v,
                 hbuf0, rowb0, fbuf0, hbuf1, rowb1, fbuf1, mbuf, shr,
                 sem0, sem1):
    c = lax.axis_index("c")
    s = lax.axis_index("s")
    pid = s // 2
    eh = s % 2
    f0 = (c * 8 + pid) * FPT
    ebase = eh * (E // EH)
    fbase = eh * (E // EH // 32)
    NP = C4 // 32
    nchunks = (E // EH) // C4
    slots = ((hbuf0, rowb0, fbuf0, sem0), (hbuf1, rowb1, fbuf1, sem1))

    def issue(ci, slot):
        hbuf, rowb, fbuf, sem = slots[slot]
        base = ebase + ci * C4
        pltpu.async_copy(row_hbm.at[pl.ds(base, C4)], rowb, sem)
        pltpu.async_copy(dupf_hbm.at[pl.ds(fbase + ci * NP, NP)],
                         fbuf.at[pl.ds(0, NP)], sem)
        for f in range(FPT):
            pltpu.async_copy(hT_hbm.at[f0 + f, pl.ds(base, C4)],
                             hbuf.at[pl.ds(f * C4, C4)], sem)

    def wait(ci, slot):
        hbuf, rowb, fbuf, sem = slots[slot]
        base = ebase + ci * C4
        pltpu.make_async_copy(row_hbm.at[pl.ds(base, C4)], rowb, sem).wait()
        pltpu.make_async_copy(dupf_hbm.at[pl.ds(fbase + ci * NP, NP)],
                              fbuf.at[pl.ds(0, NP)], sem).wait()
        for f in range(FPT):
            pltpu.make_async_copy(hT_hbm.at[f0 + f, pl.ds(base, C4)],
                                  hbuf.at[pl.ds(f * C4, C4)], sem).wait()

    issue(0, 0)
    issue(1, 1)

    neg = jnp.full((L,), _NEG_INF, jnp.float32)

    def initloop(i, carry):
        agg_v[pl.ds(i * L, L)] = neg
        return carry

    lax.fori_loop(0, (FPT * N) // L, initloop, 0)

    def process(hbuf, rowb, fbuf):
        def load_group(k):
            sl = pl.ds(k * L, L)
            rv = rowb[sl]
            hvs = []
            for f in range(FPT):
                hvs.append(hbuf[pl.ds(f * C4 + k * L, L)])
            return rv, hvs

        def fast_group(rv, hvs):
            ris = [rv + (f * N) for f in range(FPT)]
            curs = [plsc.load_gather(agg_v, [ris[f]]) for f in range(FPT)]
            for f in range(FPT):
                plsc.store_scatter(agg_v, [ris[f]],
                                   jnp.maximum(curs[f], hvs[f]))

        def slow_group(rv, hvs):
            for f in range(FPT):
                ri = rv + (f * N)
                hv = hvs[f]

                def cond(m):
                    return jnp.any(m)

                def body(m):
                    cur = plsc.load_gather(agg_v, [ri])
                    val = jnp.maximum(cur, hv)
                    plsc.store_scatter(agg_v, [ri], val, mask=m)
                    chk = plsc.load_gather(agg_v, [ri])
                    return m & (chk < hv)

                lax.while_loop(cond, body, jnp.ones((L,), jnp.bool_))

        def group2(g, c2):
            rv0, hvs0 = load_group(2 * g)
            rv1, hvs1 = load_group(2 * g + 1)
            fv = fbuf[pl.ds(g, L)]
            has_dup = fv[0] != 0

            def fast(_):
                fast_group(rv0, hvs0)
                fast_group(rv1, hvs1)
                return 0

            def slow(_):
                slow_group(rv0, hvs0)
                slow_group(rv1, hvs1)
                return 0

            lax.cond(has_dup, slow, fast, 0)
            return c2

        lax.fori_loop(0, C4 // L // 2, group2, 0)

    def pair(i, carry):
        for slot in (0, 1):
            ci = 2 * i + slot
            wait(ci, slot)
            hbuf, rowb, fbuf, _ = slots[slot]
            process(hbuf, rowb, fbuf)

            @pl.when(ci + 2 < nchunks)
            def _():
                issue(ci + 2, slot)
        return carry

    lax.fori_loop(0, nchunks // 2, pair, 0)
    wait(nchunks - 1, 0)
    process(hbuf0, rowb0, fbuf0)

    def mergeloop(j, carry):
        @pl.when(eh == 1)
        def _():
            pltpu.sync_copy(agg_v.at[pl.ds(j * MP, MP)], shr.at[pid])

        plsc.subcore_barrier()

        @pl.when(eh == 0)
        def _():
            pltpu.sync_copy(shr.at[pid], mbuf)

            def vloop(i, c2):
                slc = pl.ds(j * MP + i * L, L)
                agg_v[slc] = jnp.maximum(agg_v[slc], mbuf[pl.ds(i * L, L)])
                return c2

            lax.fori_loop(0, MP // L, vloop, 0)

        plsc.subcore_barrier()
        return carry

    lax.fori_loop(0, (FPT * N) // MP, mergeloop, 0)

    @pl.when(eh == 0)
    def _():
        def fixloop(i, carry):
            sl = pl.ds(i * L, L)
            v = agg_v[sl]
            ok = (v - v) == 0.0
            agg_v[sl] = jnp.where(ok, v, 0.0)
            return carry

        lax.fori_loop(0, (FPT * N) // L, fixloop, 0)
        for f in range(FPT):
            pltpu.sync_copy(agg_v.at[pl.ds(f * N, N)], aggT_hbm.at[f0 + f])


def _k4(hT, row, dupf):
    fn = pl.kernel(
        _segmax_body,
        out_type=jax.ShapeDtypeStruct((D, N), jnp.float32),
        mesh=_sc_mesh(),
        compiler_params=_SC_PARAMS,
        scratch_types=[
            pltpu.VMEM((FPT * N,), jnp.float32),
            pltpu.VMEM((FPT * C4,), jnp.float32),
            pltpu.VMEM((C4,), jnp.int32),
            pltpu.VMEM((C4 // 32 + L,), jnp.int32),
            pltpu.VMEM((FPT * C4,), jnp.float32),
            pltpu.VMEM((C4,), jnp.int32),
            pltpu.VMEM((C4 // 32 + L,), jnp.int32),
            pltpu.VMEM((MP,), jnp.float32),
            pltpu.VMEM_SHARED((8, MP), jnp.float32),
            pltpu.SemaphoreType.DMA,
            pltpu.SemaphoreType.DMA,
        ],
    )
    return fn(hT, row, dupf)


def kernel(x, edge_index, edge_attr, glob, batch,
           W_edge, b_edge, W_node, b_node, W_node2, b_node2, W_glob, b_glob):
    row = edge_index[0]
    col = edge_index[1]
    Wex, Wea = W_edge[:, :D], W_edge[:, D:]
    Wnx, Wne = W_node[:, :D], W_node[:, D:]
    W2a, W2g = W_node2[:, :D], W_node2[:, D:]
    Wgg, Wgm = W_glob[:, :32], W_glob[:, 32:]
    be2d = b_edge.reshape(1, D)
    bn2d = b_node.reshape(D, 1)
    b22d = b_node2.reshape(1, D)
    bg2d = b_glob.reshape(1, 32)
    batch2d = batch.reshape(N, 1)

    rowg = row.reshape(E // 16, 16)
    r, dupf2d, q, p = _k1b(edge_attr, Wea, be2d, rowg, x, Wex, Wnx)
    dupf = dupf2d.reshape(E // 32)
    e, pc = _k2(q, p, row, col, r)
    hT = _k3(e, pc, Wne, bn2d)
    aggT = _k4(hT, row, dupf)
    xn, u = _k5(aggT, batch2d, glob, W2a, b22d, W2g, Wgg, Wgm, bg2d)
    return (xn, e, u)


# speedup vs baseline: 4.6381x; 1.0001x over previous
import jax
import jax.numpy as jnp
from jax import lax
from jax.experimental import pallas as pl
from jax.experimental.pallas import tpu as pltpu
from jax.experimental.pallas import tpu_sc as plsc

N = 10000
E = 320000
D = 128
NB = 16
NC = 2
NS = 16
NW = NC * NS
L = 16

BE = 6400
C2 = 40
C4 = 1280
FPT = 8
EH = 2
MP = 10000

_NEG_INF = float("-inf")


def _r_body(ea_ref, wea_ref, be_ref, rowg_ref, x_ref, wex_ref, wnx_ref,
            r_ref, flag_ref, q_ref, p_ref):
    @pl.when(pl.program_id(0) == 0)
    def _():
        xb = x_ref[...]
        q_ref[...] = lax.dot_general(xb, wex_ref[...], (((1,), (1,)), ((), ())),
                                     preferred_element_type=jnp.float32)
        p_ref[...] = lax.dot_general(xb, wnx_ref[...], (((1,), (1,)), ((), ())),
                                     preferred_element_type=jnp.float32)

    r_ref[...] = lax.dot_general(ea_ref[...], wea_ref[...],
                                 (((1,), (1,)), ((), ())),
                                 preferred_element_type=jnp.float32) + be_ref[...]
    rg = rowg_ref[...]
    dup = None
    for sh in range(1, 16):
        c = jnp.concatenate([rg[:, sh:], rg[:, :sh]], axis=1)
        m = rg == c
        dup = m if dup is None else (dup | m)
    dupg = jnp.any(dup, axis=1)
    pairs = dupg.reshape(BE // 32, 2)
    flag_ref[...] = (pairs[:, 0] | pairs[:, 1]).astype(jnp.int32) \
        .reshape(BE // 32, 1)


def _k1b(edge_attr, Wea, be2d, rowg, x, Wex, Wnx):
    return pl.pallas_call(
        _r_body,
        grid=(E // BE,),
        in_specs=[
            pl.BlockSpec((BE, 16), lambda i: (i, 0)),
            pl.BlockSpec((D, 16), lambda i: (0, 0)),
            pl.BlockSpec((1, D), lambda i: (0, 0)),
            pl.BlockSpec((BE // 16, 16), lambda i: (i, 0)),
            pl.BlockSpec((N, D), lambda i: (0, 0)),
            pl.BlockSpec((D, D), lambda i: (0, 0)),
            pl.BlockSpec((D, D), lambda i: (0, 0)),
        ],
        out_specs=[
            pl.BlockSpec((BE, D), lambda i: (i, 0)),
            pl.BlockSpec((BE // 32, 1), lambda i: (i, 0)),
            pl.BlockSpec((N, D), lambda i: (0, 0)),
            pl.BlockSpec((N, D), lambda i: (0, 0)),
        ],
        out_shape=[
            jax.ShapeDtypeStruct((E, D), jnp.float32),
            jax.ShapeDtypeStruct((E // 32, 1), jnp.int32),
            jax.ShapeDtypeStruct((N, D), jnp.float32),
            jax.ShapeDtypeStruct((N, D), jnp.float32),
        ],
    )(edge_attr, Wea, be2d, rowg, x, Wex, Wnx)


def _hT_body(e_ref, pc_ref, wne_ref, bn_ref, hT_ref):
    hT_ref[...] = (lax.dot_general(wne_ref[...], e_ref[...],
                                   (((1,), (1,)), ((), ())),
                                   preferred_element_type=jnp.float32)
                   + lax.transpose(pc_ref[...], (1, 0)) + bn_ref[...])


def _k3(e, pc, Wne, bn2d):
    return pl.pallas_call(
        _hT_body,
        grid=(E // BE,),
        in_specs=[
            pl.BlockSpec((BE, D), lambda i: (i, 0)),
            pl.BlockSpec((BE, D), lambda i: (i, 0)),
            pl.BlockSpec((D, D), lambda i: (0, 0)),
            pl.BlockSpec((D, 1), lambda i: (0, 0)),
        ],
        out_specs=pl.BlockSpec((D, BE), lambda i: (0, i)),
        out_shape=jax.ShapeDtypeStruct((D, E), jnp.float32),
    )(e, pc, Wne, bn2d)


def _final_body(aggT_ref, b2d_ref, glob_ref, w2a_ref, b2_ref, w2g_ref,
                wgg_ref, wgm_ref, bg_ref, xn_ref, u_ref):
    aggT = aggT_ref[...]
    xb = lax.dot_general(aggT, w2a_ref[...], (((0,), (1,)), ((), ())),
                         preferred_element_type=jnp.float32)
    oh = (b2d_ref[...] == lax.broadcasted_iota(jnp.int32, (N, NB), 1))
    oh = oh.astype(jnp.float32)
    G2 = lax.dot_general(glob_ref[...], w2g_ref[...], (((1,), (1,)), ((), ())),
                         preferred_element_type=jnp.float32)
    xn = xb + lax.dot_general(oh, G2, (((1,), (0,)), ((), ())),
                              preferred_element_type=jnp.float32) + b2_ref[...]
    xn = jnp.maximum(xn, 0.0)
    xn_ref[...] = xn

    S = lax.dot_general(oh, xn, (((0,), (0,)), ((), ())),
                        preferred_element_type=jnp.float32)
    cnt = lax.dot_general(oh, jnp.ones((N, D), jnp.float32),
                          (((0,), (0,)), ((), ())),
                          preferred_element_type=jnp.float32)
    mean = S / jnp.maximum(cnt, 1.0)
    u = (lax.dot_general(glob_ref[...], wgg_ref[...],
                         (((1,), (1,)), ((), ())),
                         preferred_element_type=jnp.float32)
         + lax.dot_general(mean, wgm_ref[...], (((1,), (1,)), ((), ())),
                           preferred_element_type=jnp.float32)
         + bg_ref[...])
    u_ref[...] = jnp.maximum(u, 0.0)


def _k5(aggT, batch2d, glob, W2a, b22d, W2g, Wgg, Wgm, bg2d):
    return pl.pallas_call(
        _final_body,
        out_shape=[
            jax.ShapeDtypeStruct((N, D), jnp.float32),
            jax.ShapeDtypeStruct((NB, 32), jnp.float32),
        ],
    )(aggT, batch2d, glob, W2a, b22d, W2g, Wgg, Wgm, bg2d)


def _sc_mesh():
    return plsc.VectorSubcoreMesh(core_axis_name="c", subcore_axis_name="s")


_SC_PARAMS = pltpu.CompilerParams(needs_layout_passes=False,
                                  use_tc_tiling_on_sc=False)


def _edge_sc_body(q_hbm, p_hbm, row_hbm, col_hbm, r_hbm, e_hbm, pc_hbm,
                  idxr0, idxc0, qrows0, prows0, rbuf0,
                  idxr1, idxc1, qrows1, prows1, rbuf1,
                  semi0, semg0, semo0, semi1, semg1, semo1):
    wid = lax.axis_index("s") * NC + lax.axis_index("c")
    base0 = wid * (E // NW)
    nchunks = (E // NW) // C2
    slots = ((idxr0, idxc0, qrows0, prows0, rbuf0, semi0, semg0, semo0),
             (idxr1, idxc1, qrows1, prows1, rbuf1, semi1, semg1, semo1))

    def issue_idx(ci, s):
        idxr, idxc, _, _, _, semi, _, _ = slots[s]
        base = base0 + ci * C2
        pltpu.async_copy(row_hbm.at[pl.ds(base, C2)], idxr, semi)
        pltpu.async_copy(col_hbm.at[pl.ds(base, C2)], idxc, semi)

    def wait_idx(ci, s):
        idxr, idxc, _, _, _, semi, _, _ = slots[s]
        base = base0 + ci * C2
        pltpu.make_async_copy(row_hbm.at[pl.ds(base, C2)], idxr, semi).wait()
        pltpu.make_async_copy(col_hbm.at[pl.ds(base, C2)], idxc, semi).wait()

    def issue_g(ci, s):
        idxr, idxc, qrows, prows, rbuf, _, semg, _ = slots[s]
        base = base0 + ci * C2
        pltpu.async_copy(q_hbm.at[idxr], qrows, semg)
        pltpu.async_copy(p_hbm.at[idxc], prows, semg)
        pltpu.async_copy(r_hbm.at[pl.ds(base, C2)], rbuf, semg)

    def wait_g(ci, s):
        idxr, idxc, qrows, prows, rbuf, _, semg, _ = slots[s]
        base = base0 + ci * C2
        pltpu.make_async_copy(q_hbm.at[idxr], qrows, semg).wait()
        pltpu.make_async_copy(p_hbm.at[idxc], prows, semg).wait()
        pltpu.make_async_copy(r_hbm.at[pl.ds(base, C2)], rbuf, semg).wait()

    def issue_out(ci, s):
        _, _, qrows, prows, _, _, _, semo = slots[s]
        base = base0 + ci * C2
        pltpu.async_copy(qrows, e_hbm.at[pl.ds(base, C2)], semo)
        pltpu.async_copy(prows, pc_hbm.at[pl.ds(base, C2)], semo)

    def wait_out(ci, s):
        _, _, qrows, prows, _, _, _, semo = slots[s]
        base = base0 + ci * C2
        pltpu.make_async_copy(qrows, e_hbm.at[pl.ds(base, C2)], semo).wait()
        pltpu.make_async_copy(prows, pc_hbm.at[pl.ds(base, C2)], semo).wait()

    issue_idx(0, 0)
    issue_idx(1, 1)
    wait_idx(0, 0)
    issue_g(0, 0)

    def pair(i, carry):
        for s in (0, 1):
            ci = 2 * i + s
            so = 1 - s

            @pl.when((ci + 1 < nchunks) & (ci >= 1))
            def _():
                wait_out(ci - 1, so)

            @pl.when(ci + 1 < nchunks)
            def _():
                wait_idx(ci + 1, so)
                issue_g(ci + 1, so)

            wait_g(ci, s)
            _, _, qrows, prows, rbuf, _, _, _ = slots[s]

            def rowloop(ii, c2):
                for j in range(D // L):
                    slc = pl.ds(j * L, L)
                    v = qrows[ii, slc] + rbuf[ii, slc]
                    qrows[ii, slc] = jnp.maximum(v, 0.0)
                return c2

            lax.fori_loop(0, C2, rowloop, 0)
            issue_out(ci, s)

            @pl.when(ci + 2 < nchunks)
            def _():
                issue_idx(ci + 2, s)
        return carry

    lax.fori_loop(0, nchunks // 2, pair, 0)
    wait_out(nchunks - 2, 0)
    wait_out(nchunks - 1, 1)


def _k2(q, p, row, col, r):
    fn = pl.kernel(
        _edge_sc_body,
        out_type=[
            jax.ShapeDtypeStruct((E, D), jnp.float32),
            jax.ShapeDtypeStruct((E, D), jnp.float32),
        ],
        mesh=_sc_mesh(),
        compiler_params=_SC_PARAMS,
        scratch_types=[
            pltpu.VMEM((C2,), jnp.int32),
            pltpu.VMEM((C2,), jnp.int32),
            pltpu.VMEM((C2, D), jnp.float32),
            pltpu.VMEM((C2, D), jnp.float32),
            pltpu.VMEM((C2, D), jnp.float32),
            pltpu.VMEM((C2,), jnp.int32),
            pltpu.VMEM((C2,), jnp.int32),
            pltpu.VMEM((C2, D), jnp.float32),
            pltpu.VMEM((C2, D), jnp.float32),
            pltpu.VMEM((C2, D), jnp.float32),
            pltpu.SemaphoreType.DMA,
            pltpu.SemaphoreType.DMA,
            pltpu.SemaphoreType.DMA,
            pltpu.SemaphoreType.DMA,
            pltpu.SemaphoreType.DMA,
            pltpu.SemaphoreType.DMA,
        ],
    )
    return fn(q, p, row, col, r)


def _segmax_body(hT_hbm, row_hbm, dupf_hbm, aggT_hbm, agg_v,
                 hbuf0, rowb0, fbuf0, hbuf1, rowb1, fbuf1, mbuf, shr,
                 sem0, sem1):
    c = lax.axis_index("c")
    s = lax.axis_index("s")
    pid = s // 2
    eh = s % 2
    f0 = (c * 8 + pid) * FPT
    ebase = eh * (E // EH)
    fbase = eh * (E // EH // 32)
    NP = C4 // 32
    nchunks = (E // EH) // C4
    slots = ((hbuf0, rowb0, fbuf0, sem0), (hbuf1, rowb1, fbuf1, sem1))

    def issue(ci, slot):
        hbuf, rowb, fbuf, sem = slots[slot]
        base = ebase + ci * C4
        pltpu.async_copy(row_hbm.at[pl.ds(base, C4)], rowb, sem)
        pltpu.async_copy(dupf_hbm.at[pl.ds(fbase + ci * NP, NP)],
                         fbuf.at[pl.ds(0, NP)], sem)
        for f in range(FPT):
            pltpu.async_copy(hT_hbm.at[f0 + f, pl.ds(base, C4)],
                             hbuf.at[pl.ds(f * C4, C4)], sem)

    def wait(ci, slot):
        hbuf, rowb, fbuf, sem = slots[slot]
        base = ebase + ci * C4
        pltpu.make_async_copy(row_hbm.at[pl.ds(base, C4)], rowb, sem).wait()
        pltpu.make_async_copy(dupf_hbm.at[pl.ds(fbase + ci * NP, NP)],
                              fbuf.at[pl.ds(0, NP)], sem).wait()
        for f in range(FPT):
            pltpu.make_async_copy(hT_hbm.at[f0 + f, pl.ds(base, C4)],
                                  hbuf.at[pl.ds(f * C4, C4)], sem).wait()

    issue(0, 0)
    issue(1, 1)

    neg = jnp.full((L,), _NEG_INF, jnp.float32)

    def initloop(i, carry):
        agg_v[pl.ds(i * L, L)] = neg
        return carry

    lax.fori_loop(0, (FPT * N) // L, initloop, 0)

    def process(hbuf, rowb, fbuf):
        def load_group(k):
            sl = pl.ds(k * L, L)
            rv = rowb[sl]
            hvs = []
            for f in range(FPT):
                hvs.append(hbuf[pl.ds(f * C4 + k * L, L)])
            return rv, hvs

        def fast_group(rv, hvs):
            ris = [rv + (f * N) for f in range(FPT)]
            curs = [plsc.load_gather(agg_v, [ris[f]]) for f in range(FPT)]
            for f in range(FPT):
                plsc.store_scatter(agg_v, [ris[f]],
                                   jnp.maximum(curs[f], hvs[f]))

        def slow_group(rv, hvs):
            for f in range(FPT):
                ri = rv + (f * N)
                hv = hvs[f]

                def cond(m):
                    return jnp.any(m)

                def body(m):
                    cur = plsc.load_gather(agg_v, [ri])
                    val = jnp.maximum(cur, hv)
                    plsc.store_scatter(agg_v, [ri], val, mask=m)
                    chk = plsc.load_gather(agg_v, [ri])
                    return m & (chk < hv)

                lax.while_loop(cond, body, jnp.ones((L,), jnp.bool_))

        def group2(g, c2):
            rv0, hvs0 = load_group(2 * g)
            rv1, hvs1 = load_group(2 * g + 1)
            fv = fbuf[pl.ds(g, L)]
            has_dup = fv[0] != 0

            def fast(_):
                fast_group(rv0, hvs0)
                fast_group(rv1, hvs1)
                return 0

            def slow(_):
                slow_group(rv0, hvs0)
                slow_group(rv1, hvs1)
                return 0

            lax.cond(has_dup, slow, fast, 0)
            return c2

        lax.fori_loop(0, C4 // L // 2, group2, 0)

    def pair(i, carry):
        for slot in (0, 1):
            ci = 2 * i + slot
            wait(ci, slot)
            hbuf, rowb, fbuf, _ = slots[slot]
            process(hbuf, rowb, fbuf)

            @pl.when(ci + 2 < nchunks)
            def _():
                issue(ci + 2, slot)
        return carry

    lax.fori_loop(0, nchunks // 2, pair, 0)
    wait(nchunks - 1, 0)
    process(hbuf0, rowb0, fbuf0)

    def mergeloop(j, carry):
        @pl.when(eh == 1)
        def _():
            pltpu.sync_copy(agg_v.at[pl.ds(j * MP, MP)], shr.at[pid])

        plsc.subcore_barrier()

        @pl.when(eh == 0)
        def _():
            pltpu.sync_copy(shr.at[pid], mbuf)

            def vloop(i, c2):
                slc = pl.ds(j * MP + i * L, L)
                agg_v[slc] = jnp.maximum(agg_v[slc], mbuf[pl.ds(i * L, L)])
                return c2

            lax.fori_loop(0, MP // L, vloop, 0)

        plsc.subcore_barrier()
        return carry

    lax.fori_loop(0, (FPT * N) // MP, mergeloop, 0)

    @pl.when(eh == 0)
    def _():
        def fixloop(i, carry):
            sl = pl.ds(i * L, L)
            v = agg_v[sl]
            ok = (v - v) == 0.0
            agg_v[sl] = jnp.where(ok, v, 0.0)
            return carry

        lax.fori_loop(0, (FPT * N) // L, fixloop, 0)
        for f in range(FPT):
            pltpu.sync_copy(agg_v.at[pl.ds(f * N, N)], aggT_hbm.at[f0 + f])


def _k4(hT, row, dupf):
    fn = pl.kernel(
        _segmax_body,
        out_type=jax.ShapeDtypeStruct((D, N), jnp.float32),
        mesh=_sc_mesh(),
        compiler_params=_SC_PARAMS,
        scratch_types=[
            pltpu.VMEM((FPT * N,), jnp.float32),
            pltpu.VMEM((FPT * C4,), jnp.float32),
            pltpu.VMEM((C4,), jnp.int32),
            pltpu.VMEM((C4 // 32 + L,), jnp.int32),
            pltpu.VMEM((FPT * C4,), jnp.float32),
            pltpu.VMEM((C4,), jnp.int32),
            pltpu.VMEM((C4 // 32 + L,), jnp.int32),
            pltpu.VMEM((MP,), jnp.float32),
            pltpu.VMEM_SHARED((8, MP), jnp.float32),
            pltpu.SemaphoreType.DMA,
            pltpu.SemaphoreType.DMA,
        ],
    )
    return fn(hT, row, dupf)


def kernel(x, edge_index, edge_attr, glob, batch,
           W_edge, b_edge, W_node, b_node, W_node2, b_node2, W_glob, b_glob):
    row = edge_index[0]
    col = edge_index[1]
    Wex, Wea = W_edge[:, :D], W_edge[:, D:]
    Wnx, Wne = W_node[:, :D], W_node[:, D:]
    W2a, W2g = W_node2[:, :D], W_node2[:, D:]
    Wgg, Wgm = W_glob[:, :32], W_glob[:, 32:]
    be2d = b_edge.reshape(1, D)
    bn2d = b_node.reshape(D, 1)
    b22d = b_node2.reshape(1, D)
    bg2d = b_glob.reshape(1, 32)
    batch2d = batch.reshape(N, 1)

    rowg = row.reshape(E // 16, 16)
    r, dupf2d, q, p = _k1b(edge_attr, Wea, be2d, rowg, x, Wex, Wnx)
    dupf = dupf2d.reshape(E // 32)
    e, pc = _k2(q, p, row, col, r)
    hT = _k3(e, pc, Wne, bn2d)
    aggT = _k4(hT, row, dupf)
    xn, u = _k5(aggT, batch2d, glob, W2a, b22d, W2g, Wgg, Wgm, bg2d)
    return (xn, e, u)


# speedup vs baseline: 4.8018x; 1.0353x over previous
import jax
import jax.numpy as jnp
from jax import lax
from jax.experimental import pallas as pl
from jax.experimental.pallas import tpu as pltpu
from jax.experimental.pallas import tpu_sc as plsc

N = 10000
E = 320000
D = 128
NB = 16
NC = 2
NS = 16
NW = NC * NS
L = 16

BE = 6400
C2 = 80
C4 = 1280
FPT = 8
EH = 2
MP = 10000

_NEG_INF = float("-inf")


def _r_body(ea_ref, wea_ref, be_ref, rowg_ref, x_ref, wex_ref, wnx_ref,
            r_ref, flag_ref, q_ref, p_ref):
    @pl.when(pl.program_id(0) == 0)
    def _():
        xb = x_ref[...]
        q_ref[...] = lax.dot_general(xb, wex_ref[...], (((1,), (1,)), ((), ())),
                                     preferred_element_type=jnp.float32)
        p_ref[...] = lax.dot_general(xb, wnx_ref[...], (((1,), (1,)), ((), ())),
                                     preferred_element_type=jnp.float32)

    r_ref[...] = lax.dot_general(ea_ref[...], wea_ref[...],
                                 (((1,), (1,)), ((), ())),
                                 preferred_element_type=jnp.float32) + be_ref[...]
    rg = rowg_ref[...]
    dup = None
    for sh in range(1, 16):
        c = jnp.concatenate([rg[:, sh:], rg[:, :sh]], axis=1)
        m = rg == c
        dup = m if dup is None else (dup | m)
    dupg = jnp.any(dup, axis=1)
    pairs = dupg.reshape(BE // 32, 2)
    flag_ref[...] = (pairs[:, 0] | pairs[:, 1]).astype(jnp.int32) \
        .reshape(BE // 32, 1)


def _k1b(edge_attr, Wea, be2d, rowg, x, Wex, Wnx):
    return pl.pallas_call(
        _r_body,
        grid=(E // BE,),
        in_specs=[
            pl.BlockSpec((BE, 16), lambda i: (i, 0)),
            pl.BlockSpec((D, 16), lambda i: (0, 0)),
            pl.BlockSpec((1, D), lambda i: (0, 0)),
            pl.BlockSpec((BE // 16, 16), lambda i: (i, 0)),
            pl.BlockSpec((N, D), lambda i: (0, 0)),
            pl.BlockSpec((D, D), lambda i: (0, 0)),
            pl.BlockSpec((D, D), lambda i: (0, 0)),
        ],
        out_specs=[
            pl.BlockSpec((BE, D), lambda i: (i, 0)),
            pl.BlockSpec((BE // 32, 1), lambda i: (i, 0)),
            pl.BlockSpec((N, D), lambda i: (0, 0)),
            pl.BlockSpec((N, D), lambda i: (0, 0)),
        ],
        out_shape=[
            jax.ShapeDtypeStruct((E, D), jnp.float32),
            jax.ShapeDtypeStruct((E // 32, 1), jnp.int32),
            jax.ShapeDtypeStruct((N, D), jnp.float32),
            jax.ShapeDtypeStruct((N, D), jnp.float32),
        ],
    )(edge_attr, Wea, be2d, rowg, x, Wex, Wnx)


def _hT_body(e_ref, pc_ref, wne_ref, bn_ref, hT_ref):
    hT_ref[...] = (lax.dot_general(wne_ref[...], e_ref[...],
                                   (((1,), (1,)), ((), ())),
                                   preferred_element_type=jnp.float32)
                   + lax.transpose(pc_ref[...], (1, 0)) + bn_ref[...])


def _k3(e, pc, Wne, bn2d):
    return pl.pallas_call(
        _hT_body,
        grid=(E // BE,),
        in_specs=[
            pl.BlockSpec((BE, D), lambda i: (i, 0)),
            pl.BlockSpec((BE, D), lambda i: (i, 0)),
            pl.BlockSpec((D, D), lambda i: (0, 0)),
            pl.BlockSpec((D, 1), lambda i: (0, 0)),
        ],
        out_specs=pl.BlockSpec((D, BE), lambda i: (0, i)),
        out_shape=jax.ShapeDtypeStruct((D, E), jnp.float32),
    )(e, pc, Wne, bn2d)


def _final_body(aggT_ref, b2d_ref, glob_ref, w2a_ref, b2_ref, w2g_ref,
                wgg_ref, wgm_ref, bg_ref, xn_ref, u_ref):
    aggT = aggT_ref[...]
    xb = lax.dot_general(aggT, w2a_ref[...], (((0,), (1,)), ((), ())),
                         preferred_element_type=jnp.float32)
    oh = (b2d_ref[...] == lax.broadcasted_iota(jnp.int32, (N, NB), 1))
    oh = oh.astype(jnp.float32)
    G2 = lax.dot_general(glob_ref[...], w2g_ref[...], (((1,), (1,)), ((), ())),
                         preferred_element_type=jnp.float32)
    xn = xb + lax.dot_general(oh, G2, (((1,), (0,)), ((), ())),
                              preferred_element_type=jnp.float32) + b2_ref[...]
    xn = jnp.maximum(xn, 0.0)
    xn_ref[...] = xn

    S = lax.dot_general(oh, xn, (((0,), (0,)), ((), ())),
                        preferred_element_type=jnp.float32)
    cnt = lax.dot_general(oh, jnp.ones((N, D), jnp.float32),
                          (((0,), (0,)), ((), ())),
                          preferred_element_type=jnp.float32)
    mean = S / jnp.maximum(cnt, 1.0)
    u = (lax.dot_general(glob_ref[...], wgg_ref[...],
                         (((1,), (1,)), ((), ())),
                         preferred_element_type=jnp.float32)
         + lax.dot_general(mean, wgm_ref[...], (((1,), (1,)), ((), ())),
                           preferred_element_type=jnp.float32)
         + bg_ref[...])
    u_ref[...] = jnp.maximum(u, 0.0)


def _k5(aggT, batch2d, glob, W2a, b22d, W2g, Wgg, Wgm, bg2d):
    return pl.pallas_call(
        _final_body,
        out_shape=[
            jax.ShapeDtypeStruct((N, D), jnp.float32),
            jax.ShapeDtypeStruct((NB, 32), jnp.float32),
        ],
    )(aggT, batch2d, glob, W2a, b22d, W2g, Wgg, Wgm, bg2d)


def _sc_mesh():
    return plsc.VectorSubcoreMesh(core_axis_name="c", subcore_axis_name="s")


_SC_PARAMS = pltpu.CompilerParams(needs_layout_passes=False,
                                  use_tc_tiling_on_sc=False)


def _edge_sc_body(q_hbm, p_hbm, row_hbm, col_hbm, r_hbm, e_hbm, pc_hbm,
                  idxr0, idxc0, qrows0, prows0, rbuf0,
                  idxr1, idxc1, qrows1, prows1, rbuf1,
                  semi0, semg0, semo0, semi1, semg1, semo1):
    wid = lax.axis_index("s") * NC + lax.axis_index("c")
    base0 = wid * (E // NW)
    nchunks = (E // NW) // C2
    slots = ((idxr0, idxc0, qrows0, prows0, rbuf0, semi0, semg0, semo0),
             (idxr1, idxc1, qrows1, prows1, rbuf1, semi1, semg1, semo1))

    def issue_idx(ci, s):
        idxr, idxc, _, _, _, semi, _, _ = slots[s]
        base = base0 + ci * C2
        pltpu.async_copy(row_hbm.at[pl.ds(base, C2)], idxr, semi)
        pltpu.async_copy(col_hbm.at[pl.ds(base, C2)], idxc, semi)

    def wait_idx(ci, s):
        idxr, idxc, _, _, _, semi, _, _ = slots[s]
        base = base0 + ci * C2
        pltpu.make_async_copy(row_hbm.at[pl.ds(base, C2)], idxr, semi).wait()
        pltpu.make_async_copy(col_hbm.at[pl.ds(base, C2)], idxc, semi).wait()

    def issue_g(ci, s):
        idxr, idxc, qrows, prows, rbuf, _, semg, _ = slots[s]
        base = base0 + ci * C2
        pltpu.async_copy(q_hbm.at[idxr], qrows, semg)
        pltpu.async_copy(p_hbm.at[idxc], prows, semg)
        pltpu.async_copy(r_hbm.at[pl.ds(base, C2)], rbuf, semg)

    def wait_g(ci, s):
        idxr, idxc, qrows, prows, rbuf, _, semg, _ = slots[s]
        base = base0 + ci * C2
        pltpu.make_async_copy(q_hbm.at[idxr], qrows, semg).wait()
        pltpu.make_async_copy(p_hbm.at[idxc], prows, semg).wait()
        pltpu.make_async_copy(r_hbm.at[pl.ds(base, C2)], rbuf, semg).wait()

    def issue_out(ci, s):
        _, _, qrows, prows, _, _, _, semo = slots[s]
        base = base0 + ci * C2
        pltpu.async_copy(qrows, e_hbm.at[pl.ds(base, C2)], semo)
        pltpu.async_copy(prows, pc_hbm.at[pl.ds(base, C2)], semo)

    def wait_out(ci, s):
        _, _, qrows, prows, _, _, _, semo = slots[s]
        base = base0 + ci * C2
        pltpu.make_async_copy(qrows, e_hbm.at[pl.ds(base, C2)], semo).wait()
        pltpu.make_async_copy(prows, pc_hbm.at[pl.ds(base, C2)], semo).wait()

    issue_idx(0, 0)
    issue_idx(1, 1)
    wait_idx(0, 0)
    issue_g(0, 0)

    def pair(i, carry):
        for s in (0, 1):
            ci = 2 * i + s
            so = 1 - s

            @pl.when((ci + 1 < nchunks) & (ci >= 1))
            def _():
                wait_out(ci - 1, so)

            @pl.when(ci + 1 < nchunks)
            def _():
                wait_idx(ci + 1, so)
                issue_g(ci + 1, so)

            wait_g(ci, s)
            compute(s)
            issue_out(ci, s)

            @pl.when(ci + 2 < nchunks)
            def _():
                issue_idx(ci + 2, s)
        return carry

    def compute(s):
        _, _, qrows, prows, rbuf, _, _, _ = slots[s]

        def rowloop(ii, c2):
            for j in range(D // L):
                slc = pl.ds(j * L, L)
                v = qrows[ii, slc] + rbuf[ii, slc]
                qrows[ii, slc] = jnp.maximum(v, 0.0)
            return c2

        lax.fori_loop(0, C2, rowloop, 0)

    lax.fori_loop(0, nchunks // 2, pair, 0)
    wait_g(nchunks - 1, 0)
    compute(0)
    issue_out(nchunks - 1, 0)
    wait_out(nchunks - 2, 1)
    wait_out(nchunks - 1, 0)


def _k2(q, p, row, col, r):
    fn = pl.kernel(
        _edge_sc_body,
        out_type=[
            jax.ShapeDtypeStruct((E, D), jnp.float32),
            jax.ShapeDtypeStruct((E, D), jnp.float32),
        ],
        mesh=_sc_mesh(),
        compiler_params=_SC_PARAMS,
        scratch_types=[
            pltpu.VMEM((C2,), jnp.int32),
            pltpu.VMEM((C2,), jnp.int32),
            pltpu.VMEM((C2, D), jnp.float32),
            pltpu.VMEM((C2, D), jnp.float32),
            pltpu.VMEM((C2, D), jnp.float32),
            pltpu.VMEM((C2,), jnp.int32),
            pltpu.VMEM((C2,), jnp.int32),
            pltpu.VMEM((C2, D), jnp.float32),
            pltpu.VMEM((C2, D), jnp.float32),
            pltpu.VMEM((C2, D), jnp.float32),
            pltpu.SemaphoreType.DMA,
            pltpu.SemaphoreType.DMA,
            pltpu.SemaphoreType.DMA,
            pltpu.SemaphoreType.DMA,
            pltpu.SemaphoreType.DMA,
            pltpu.SemaphoreType.DMA,
        ],
    )
    return fn(q, p, row, col, r)


def _segmax_body(hT_hbm, row_hbm, dupf_hbm, aggT_hbm, agg_v,
                 hbuf0, rowb0, fbuf0, hbuf1, rowb1, fbuf1, mbuf, shr,
                 sem0, sem1):
    c = lax.axis_index("c")
    s = lax.axis_index("s")
    pid = s // 2
    eh = s % 2
    f0 = (c * 8 + pid) * FPT
    ebase = eh * (E // EH)
    fbase = eh * (E // EH // 32)
    NP = C4 // 32
    nchunks = (E // EH) // C4
    slots = ((hbuf0, rowb0, fbuf0, sem0), (hbuf1, rowb1, fbuf1, sem1))

    def issue(ci, slot):
        hbuf, rowb, fbuf, sem = slots[slot]
        base = ebase + ci * C4
        pltpu.async_copy(row_hbm.at[pl.ds(base, C4)], rowb, sem)
        pltpu.async_copy(dupf_hbm.at[pl.ds(fbase + ci * NP, NP)],
                         fbuf.at[pl.ds(0, NP)], sem)
        for f in range(FPT):
            pltpu.async_copy(hT_hbm.at[f0 + f, pl.ds(base, C4)],
                             hbuf.at[pl.ds(f * C4, C4)], sem)

    def wait(ci, slot):
        hbuf, rowb, fbuf, sem = slots[slot]
        base = ebase + ci * C4
        pltpu.make_async_copy(row_hbm.at[pl.ds(base, C4)], rowb, sem).wait()
        pltpu.make_async_copy(dupf_hbm.at[pl.ds(fbase + ci * NP, NP)],
                              fbuf.at[pl.ds(0, NP)], sem).wait()
        for f in range(FPT):
            pltpu.make_async_copy(hT_hbm.at[f0 + f, pl.ds(base, C4)],
                                  hbuf.at[pl.ds(f * C4, C4)], sem).wait()

    issue(0, 0)
    issue(1, 1)

    neg = jnp.full((L,), _NEG_INF, jnp.float32)

    def initloop(i, carry):
        agg_v[pl.ds(i * L, L)] = neg
        return carry

    lax.fori_loop(0, (FPT * N) // L, initloop, 0)

    def process(hbuf, rowb, fbuf):
        def load_group(k):
            sl = pl.ds(k * L, L)
            rv = rowb[sl]
            hvs = []
            for f in range(FPT):
                hvs.append(hbuf[pl.ds(f * C4 + k * L, L)])
            return rv, hvs

        def fast_group(rv, hvs):
            ris = [rv + (f * N) for f in range(FPT)]
            curs = [plsc.load_gather(agg_v, [ris[f]]) for f in range(FPT)]
            for f in range(FPT):
                plsc.store_scatter(agg_v, [ris[f]],
                                   jnp.maximum(curs[f], hvs[f]))

        def slow_group(rv, hvs):
            for f in range(FPT):
                ri = rv + (f * N)
                hv = hvs[f]

                def cond(m):
                    return jnp.any(m)

                def body(m):
                    cur = plsc.load_gather(agg_v, [ri])
                    val = jnp.maximum(cur, hv)
                    plsc.store_scatter(agg_v, [ri], val, mask=m)
                    chk = plsc.load_gather(agg_v, [ri])
                    return m & (chk < hv)

                lax.while_loop(cond, body, jnp.ones((L,), jnp.bool_))

        def group2(g, c2):
            rv0, hvs0 = load_group(2 * g)
            rv1, hvs1 = load_group(2 * g + 1)
            fv = fbuf[pl.ds(g, L)]
            has_dup = fv[0] != 0

            def fast(_):
                fast_group(rv0, hvs0)
                fast_group(rv1, hvs1)
                return 0

            def slow(_):
                slow_group(rv0, hvs0)
                slow_group(rv1, hvs1)
                return 0

            lax.cond(has_dup, slow, fast, 0)
            return c2

        lax.fori_loop(0, C4 // L // 2, group2, 0)

    def pair(i, carry):
        for slot in (0, 1):
            ci = 2 * i + slot
            wait(ci, slot)
            hbuf, rowb, fbuf, _ = slots[slot]
            process(hbuf, rowb, fbuf)

            @pl.when(ci + 2 < nchunks)
            def _():
                issue(ci + 2, slot)
        return carry

    lax.fori_loop(0, nchunks // 2, pair, 0)
    wait(nchunks - 1, 0)
    process(hbuf0, rowb0, fbuf0)

    def mergeloop(j, carry):
        @pl.when(eh == 1)
        def _():
            pltpu.sync_copy(agg_v.at[pl.ds(j * MP, MP)], shr.at[pid])

        plsc.subcore_barrier()

        @pl.when(eh == 0)
        def _():
            pltpu.sync_copy(shr.at[pid], mbuf)

            def vloop(i, c2):
                slc = pl.ds(j * MP + i * L, L)
                agg_v[slc] = jnp.maximum(agg_v[slc], mbuf[pl.ds(i * L, L)])
                return c2

            lax.fori_loop(0, MP // L, vloop, 0)

        plsc.subcore_barrier()
        return carry

    lax.fori_loop(0, (FPT * N) // MP, mergeloop, 0)

    @pl.when(eh == 0)
    def _():
        def fixloop(i, carry):
            sl = pl.ds(i * L, L)
            v = agg_v[sl]
            ok = (v - v) == 0.0
            agg_v[sl] = jnp.where(ok, v, 0.0)
            return carry

        lax.fori_loop(0, (FPT * N) // L, fixloop, 0)
        for f in range(FPT):
            pltpu.sync_copy(agg_v.at[pl.ds(f * N, N)], aggT_hbm.at[f0 + f])


def _k4(hT, row, dupf):
    fn = pl.kernel(
        _segmax_body,
        out_type=jax.ShapeDtypeStruct((D, N), jnp.float32),
        mesh=_sc_mesh(),
        compiler_params=_SC_PARAMS,
        scratch_types=[
            pltpu.VMEM((FPT * N,), jnp.float32),
            pltpu.VMEM((FPT * C4,), jnp.float32),
            pltpu.VMEM((C4,), jnp.int32),
            pltpu.VMEM((C4 // 32 + L,), jnp.int32),
            pltpu.VMEM((FPT * C4,), jnp.float32),
            pltpu.VMEM((C4,), jnp.int32),
            pltpu.VMEM((C4 // 32 + L,), jnp.int32),
            pltpu.VMEM((MP,), jnp.float32),
            pltpu.VMEM_SHARED((8, MP), jnp.float32),
            pltpu.SemaphoreType.DMA,
            pltpu.SemaphoreType.DMA,
        ],
    )
    return fn(hT, row, dupf)


def kernel(x, edge_index, edge_attr, glob, batch,
           W_edge, b_edge, W_node, b_node, W_node2, b_node2, W_glob, b_glob):
    row = edge_index[0]
    col = edge_index[1]
    Wex, Wea = W_edge[:, :D], W_edge[:, D:]
    Wnx, Wne = W_node[:, :D], W_node[:, D:]
    W2a, W2g = W_node2[:, :D], W_node2[:, D:]
    Wgg, Wgm = W_glob[:, :32], W_glob[:, 32:]
    be2d = b_edge.reshape(1, D)
    bn2d = b_node.reshape(D, 1)
    b22d = b_node2.reshape(1, D)
    bg2d = b_glob.reshape(1, 32)
    batch2d = batch.reshape(N, 1)

    rowg = row.reshape(E // 16, 16)
    r, dupf2d, q, p = _k1b(edge_attr, Wea, be2d, rowg, x, Wex, Wnx)
    dupf = dupf2d.reshape(E // 32)
    e, pc = _k2(q, p, row, col, r)
    hT = _k3(e, pc, Wne, bn2d)
    aggT = _k4(hT, row, dupf)
    xn, u = _k5(aggT, batch2d, glob, W2a, b22d, W2g, Wgg, Wgm, bg2d)
    return (xn, e, u)
